# initial kernel scaffold (unmeasured)
import jax
import jax.numpy as jnp
from jax import lax
from jax.experimental import pallas as pl
from jax.experimental.pallas import tpu as pltpu

N_DEV = 4
SQ = 1024
SKV = 1024
H = 8
DH = 128
D_MODEL = 1024
SCALE = 0.08838834764831843


def kernel(x, Wq, K_ext, V_ext, Wo):
    i = lax.axis_index("i")

    xb = x.astype(jnp.bfloat16)
    wq_r = Wq.reshape(D_MODEL, H, DH).transpose(1, 0, 2).astype(jnp.bfloat16)
    wo_r = Wo.reshape(H, DH, D_MODEL).astype(jnp.bfloat16)

    k_s = lax.dynamic_slice_in_dim(K_ext, 8 * i, H, axis=2)
    v_s = lax.dynamic_slice_in_dim(V_ext, 8 * i, H, axis=2)
    k_r = jnp.roll(k_s.transpose(0, 2, 1, 3), -i, axis=0).astype(jnp.bfloat16)
    v_r = jnp.roll(v_s.transpose(0, 2, 1, 3), -i, axis=0).astype(jnp.bfloat16)

    def body(x_ref, wq_ref, k_ref, v_ref, wo_ref, out_ref,
             x_full, acc, oacc, maskadd, rs_recv,
             ag_ss, ag_rs, rs_ss, rs_rs):
        my = lax.axis_index("i")
        left = (my + N_DEV - 1) % N_DEV
        right = (my + 1) % N_DEV

        barrier = pltpu.get_barrier_semaphore()
        pl.semaphore_signal(barrier, inc=1, device_id=(left,),
                            device_id_type=pl.DeviceIdType.MESH)
        pl.semaphore_signal(barrier, inc=1, device_id=(right,),
                            device_id_type=pl.DeviceIdType.MESH)
        pl.semaphore_wait(barrier, 2)

        qi = lax.broadcasted_iota(jnp.int32, (SQ, SKV), 0)
        ki = lax.broadcasted_iota(jnp.int32, (SQ, SKV), 1)
        keep = (jnp.abs(qi - ki) <= 128) | (ki < 32) | (qi < 32)
        maskadd[:, :] = jnp.where(keep, 0.0, -1e9).astype(jnp.float32)

        x_full[0] = x_ref[0]
        for h in range(N_DEV - 1):
            rdma = pltpu.make_async_remote_copy(
                src_ref=x_full.at[(N_DEV - h) % N_DEV],
                dst_ref=x_full.at[N_DEV - 1 - h],
                send_sem=ag_ss.at[h],
                recv_sem=ag_rs.at[h],
                device_id=(right,),
                device_id_type=pl.DeviceIdType.MESH,
            )
            rdma.start()
            rdma.wait()

        def batch_body(j, _):
            x_j = x_full[pl.ds(j, 1), :, :].reshape(SQ, D_MODEL)
            oacc[:, :] = jnp.zeros((SQ, D_MODEL), jnp.float32)

            def head_body(h, _):
                wqh = wq_ref[pl.ds(h, 1), :, :].reshape(D_MODEL, DH)
                q = jnp.dot(x_j, wqh, preferred_element_type=jnp.float32)
                q = (q * SCALE).astype(jnp.bfloat16)
                kh = k_ref[pl.ds(j, 1), pl.ds(h, 1), :, :].reshape(SKV, DH)
                s = lax.dot_general(
                    q, kh, (((1,), (1,)), ((), ())),
                    preferred_element_type=jnp.float32)
                s = s + maskadd[:, :]
                mx = jnp.max(s, axis=1, keepdims=True)
                w = jnp.exp(s - mx)
                den = jnp.sum(w, axis=1, keepdims=True)
                wn = (w / den).astype(jnp.bfloat16)
                vh = v_ref[pl.ds(j, 1), pl.ds(h, 1), :, :].reshape(SKV, DH)
                ctx = jnp.dot(wn, vh,
                              preferred_element_type=jnp.float32
                              ).astype(jnp.bfloat16)
                woh = wo_ref[pl.ds(h, 1), :, :].reshape(DH, D_MODEL)
                oacc[:, :] = oacc[:, :] + jnp.dot(
                    ctx, woh, preferred_element_type=jnp.float32)
                return 0

            lax.fori_loop(0, H, head_body, 0)
            acc[pl.ds(j, 1), :, :] = oacc[:, :].astype(jnp.bfloat16).reshape(
                1, SQ, D_MODEL)
            return 0

        lax.fori_loop(0, N_DEV, batch_body, 0)

        for s in range(N_DEV - 1):
            rdma = pltpu.make_async_remote_copy(
                src_ref=acc.at[N_DEV - 1 - s],
                dst_ref=rs_recv.at[s],
                send_sem=rs_ss.at[s],
                recv_sem=rs_rs.at[s],
                device_id=(right,),
                device_id_type=pl.DeviceIdType.MESH,
            )
            rdma.start()
            rdma.wait()
            tgt = N_DEV - 2 - s
            acc[tgt] = (acc[tgt].astype(jnp.float32)
                        + rs_recv[s].astype(jnp.float32)).astype(jnp.bfloat16)

        out_ref[0] = acc[0].astype(jnp.float32)

    out_shape = jax.ShapeDtypeStruct((1, SQ, D_MODEL), jnp.float32)
    return pl.pallas_call(
        body,
        out_shape=out_shape,
        in_specs=[pl.BlockSpec(memory_space=pltpu.VMEM)] * 5,
        out_specs=pl.BlockSpec(memory_space=pltpu.VMEM),
        scratch_shapes=[
            pltpu.VMEM((N_DEV, SQ, D_MODEL), jnp.bfloat16),
            pltpu.VMEM((N_DEV, SQ, D_MODEL), jnp.bfloat16),
            pltpu.VMEM((SQ, D_MODEL), jnp.float32),
            pltpu.VMEM((SQ, SKV), jnp.float32),
            pltpu.VMEM((N_DEV - 1, SQ, D_MODEL), jnp.bfloat16),
            pltpu.SemaphoreType.DMA((N_DEV - 1,)),
            pltpu.SemaphoreType.DMA((N_DEV - 1,)),
            pltpu.SemaphoreType.DMA((N_DEV - 1,)),
            pltpu.SemaphoreType.DMA((N_DEV - 1,)),
        ],
        compiler_params=pltpu.CompilerParams(collective_id=0),
    )(xb, wq_r, k_r, v_r, wo_r)


# baseline (device time: 371345 ns/iter reference)
import jax
import jax.numpy as jnp
from jax import lax
from jax.experimental import pallas as pl
from jax.experimental.pallas import tpu as pltpu

N_DEV = 4
SQ = 1024
SKV = 1024
H = 8
DH = 128
D_MODEL = 1024
SCALE = 0.08838834764831843


def kernel(x, Wq, K_ext, V_ext, Wo):
    i = lax.axis_index("i")

    xb = x.astype(jnp.bfloat16)
    wq_r = Wq.reshape(D_MODEL, H, DH).transpose(1, 0, 2).astype(jnp.bfloat16)
    wo_r = Wo.reshape(H, DH, D_MODEL).astype(jnp.bfloat16)

    k_s = lax.dynamic_slice_in_dim(K_ext, 8 * i, H, axis=2)
    v_s = lax.dynamic_slice_in_dim(V_ext, 8 * i, H, axis=2)
    k_r = jnp.roll(k_s.transpose(0, 2, 1, 3), -i, axis=0).astype(jnp.bfloat16)
    v_r = jnp.roll(v_s.transpose(0, 2, 1, 3), -i, axis=0).astype(jnp.bfloat16)

    def body(x_ref, wq_ref, k_ref, v_ref, wo_ref, out_ref,
             x_full, acc, oacc, maskadd, rs_recv,
             ag_ss, ag_rs, rs_ss, rs_rs):
        my = lax.axis_index("i")
        left = (my + N_DEV - 1) % N_DEV
        right = (my + 1) % N_DEV

        barrier = pltpu.get_barrier_semaphore()
        pl.semaphore_signal(barrier, inc=1, device_id=(left,),
                            device_id_type=pl.DeviceIdType.MESH)
        pl.semaphore_signal(barrier, inc=1, device_id=(right,),
                            device_id_type=pl.DeviceIdType.MESH)
        pl.semaphore_wait(barrier, 2)

        qi = lax.broadcasted_iota(jnp.int32, (SQ, SKV), 0)
        ki = lax.broadcasted_iota(jnp.int32, (SQ, SKV), 1)
        keep = (jnp.abs(qi - ki) <= 128) | (ki < 32) | (qi < 32)
        maskadd[:, :] = jnp.where(keep, 0.0, -1e9).astype(jnp.float32)

        x_full[0] = x_ref[0]
        for h in range(N_DEV - 1):
            rdma = pltpu.make_async_remote_copy(
                src_ref=x_full.at[(N_DEV - h) % N_DEV],
                dst_ref=x_full.at[N_DEV - 1 - h],
                send_sem=ag_ss.at[h],
                recv_sem=ag_rs.at[h],
                device_id=(right,),
                device_id_type=pl.DeviceIdType.MESH,
            )
            rdma.start()
            rdma.wait()

        def batch_body(j, _):
            x_j = x_full[pl.ds(j, 1), :, :].reshape(SQ, D_MODEL)
            oacc[:, :] = jnp.zeros((SQ, D_MODEL), jnp.float32)

            def head_body(h, _):
                wqh = wq_ref[pl.ds(h, 1), :, :].reshape(D_MODEL, DH)
                q = jnp.dot(x_j, wqh, preferred_element_type=jnp.float32)
                q = (q * SCALE).astype(jnp.bfloat16)
                kh = k_ref[pl.ds(j, 1), pl.ds(h, 1), :, :].reshape(SKV, DH)
                s = lax.dot_general(
                    q, kh, (((1,), (1,)), ((), ())),
                    preferred_element_type=jnp.float32)
                s = s + maskadd[:, :]
                mx = jnp.max(s, axis=1, keepdims=True)
                w = jnp.exp(s - mx)
                den = jnp.sum(w, axis=1, keepdims=True)
                wn = (w / den).astype(jnp.bfloat16)
                vh = v_ref[pl.ds(j, 1), pl.ds(h, 1), :, :].reshape(SKV, DH)
                ctx = jnp.dot(wn, vh,
                              preferred_element_type=jnp.float32
                              ).astype(jnp.bfloat16)
                woh = wo_ref[pl.ds(h, 1), :, :].reshape(DH, D_MODEL)
                oacc[:, :] = oacc[:, :] + jnp.dot(
                    ctx, woh, preferred_element_type=jnp.float32)
                return 0

            lax.fori_loop(0, H, head_body, 0)
            acc[pl.ds(j, 1), :, :] = oacc[:, :].astype(jnp.bfloat16).reshape(
                1, SQ, D_MODEL)
            return 0

        lax.fori_loop(0, N_DEV, batch_body, 0)

        for s in range(N_DEV - 1):
            rdma = pltpu.make_async_remote_copy(
                src_ref=acc.at[N_DEV - 1 - s],
                dst_ref=rs_recv.at[s],
                send_sem=rs_ss.at[s],
                recv_sem=rs_rs.at[s],
                device_id=(right,),
                device_id_type=pl.DeviceIdType.MESH,
            )
            rdma.start()
            rdma.wait()
            tgt = N_DEV - 2 - s
            acc[tgt] = (acc[tgt].astype(jnp.float32)
                        + rs_recv[s].astype(jnp.float32)).astype(jnp.bfloat16)

        out_ref[0] = acc[0].astype(jnp.float32)

    out_shape = jax.ShapeDtypeStruct((1, SQ, D_MODEL), jnp.float32)
    return pl.pallas_call(
        body,
        out_shape=out_shape,
        in_specs=[pl.BlockSpec(memory_space=pltpu.VMEM)] * 5,
        out_specs=pl.BlockSpec(memory_space=pltpu.VMEM),
        scratch_shapes=[
            pltpu.VMEM((N_DEV, SQ, D_MODEL), jnp.bfloat16),
            pltpu.VMEM((N_DEV, SQ, D_MODEL), jnp.bfloat16),
            pltpu.VMEM((SQ, D_MODEL), jnp.float32),
            pltpu.VMEM((SQ, SKV), jnp.float32),
            pltpu.VMEM((N_DEV - 1, SQ, D_MODEL), jnp.bfloat16),
            pltpu.SemaphoreType.DMA((N_DEV - 1,)),
            pltpu.SemaphoreType.DMA((N_DEV - 1,)),
            pltpu.SemaphoreType.DMA((N_DEV - 1,)),
            pltpu.SemaphoreType.DMA((N_DEV - 1,)),
        ],
        compiler_params=pltpu.CompilerParams(
            collective_id=0, vmem_limit_bytes=100 * 1024 * 1024),
    )(xb, wq_r, k_r, v_r, wo_r)


# device time: 249153 ns/iter; 1.4904x vs baseline; 1.4904x over previous
import jax
import jax.numpy as jnp
from jax import lax
from jax.experimental import pallas as pl
from jax.experimental.pallas import tpu as pltpu

N_DEV = 4
SQ = 1024
SKV = 1024
H = 8
DH = 128
D_MODEL = 1024
SCALE = 0.08838834764831843


def kernel(x, Wq, K_ext, V_ext, Wo):
    i = lax.axis_index("i")

    xb = x.astype(jnp.bfloat16)
    wq_r = Wq.reshape(D_MODEL, H, DH).transpose(1, 0, 2).astype(jnp.bfloat16)
    wo_r = Wo.reshape(H, DH, D_MODEL).astype(jnp.bfloat16)

    k_s = lax.dynamic_slice_in_dim(K_ext, 8 * i, H, axis=2)
    v_s = lax.dynamic_slice_in_dim(V_ext, 8 * i, H, axis=2)
    k_r = jnp.roll(k_s.transpose(0, 2, 1, 3), -i, axis=0).astype(jnp.bfloat16)
    v_r = jnp.roll(v_s.transpose(0, 2, 1, 3), -i, axis=0).astype(jnp.bfloat16)

    def body(x_ref, wq_ref, k_ref, v_ref, wo_ref, out_ref,
             x_full, acc, oacc, maskadd, rs_recv,
             ag_ss, ag_rs, rs_ss, rs_rs):
        my = lax.axis_index("i")
        left = (my + N_DEV - 1) % N_DEV
        right = (my + 1) % N_DEV

        barrier = pltpu.get_barrier_semaphore()
        pl.semaphore_signal(barrier, inc=1, device_id=(left,),
                            device_id_type=pl.DeviceIdType.MESH)
        pl.semaphore_signal(barrier, inc=1, device_id=(right,),
                            device_id_type=pl.DeviceIdType.MESH)
        pl.semaphore_wait(barrier, 2)

        qi = lax.broadcasted_iota(jnp.int32, (SQ, SKV), 0)
        ki = lax.broadcasted_iota(jnp.int32, (SQ, SKV), 1)
        keep = (jnp.abs(qi - ki) <= 128) | (ki < 32) | (qi < 32)
        maskadd[:, :] = jnp.where(keep, 0.0, -1e9).astype(jnp.float32)

        x_full[0] = x_ref[0]

        ag = [
            pltpu.make_async_remote_copy(
                src_ref=x_full.at[(N_DEV - h) % N_DEV],
                dst_ref=x_full.at[N_DEV - 1 - h],
                send_sem=ag_ss.at[h],
                recv_sem=ag_rs.at[h],
                device_id=(right,),
                device_id_type=pl.DeviceIdType.MESH,
            )
            for h in range(N_DEV - 1)
        ]
        rs = [
            pltpu.make_async_remote_copy(
                src_ref=acc.at[N_DEV - 1 - s],
                dst_ref=rs_recv.at[s],
                send_sem=rs_ss.at[s],
                recv_sem=rs_rs.at[s],
                device_id=(right,),
                device_id_type=pl.DeviceIdType.MESH,
            )
            for s in range(N_DEV - 1)
        ]

        def compute_slot(j):
            x_j = x_full[j]
            oacc[:, :] = jnp.zeros((SQ, D_MODEL), jnp.float32)

            def head_body(h, _):
                wqh = wq_ref[pl.ds(h, 1), :, :].reshape(D_MODEL, DH)
                q = jnp.dot(x_j, wqh, preferred_element_type=jnp.float32)
                q = (q * SCALE).astype(jnp.bfloat16)
                kh = k_ref[j, pl.ds(h, 1), :, :].reshape(SKV, DH)
                s = lax.dot_general(
                    q, kh, (((1,), (1,)), ((), ())),
                    preferred_element_type=jnp.float32)
                s = s + maskadd[:, :]
                mx = jnp.max(s, axis=1, keepdims=True)
                w = jnp.exp(s - mx)
                den = jnp.sum(w, axis=1, keepdims=True)
                vh = v_ref[j, pl.ds(h, 1), :, :].reshape(SKV, DH)
                ctx = jnp.dot(w.astype(jnp.bfloat16), vh,
                              preferred_element_type=jnp.float32)
                ctx = (ctx * (1.0 / den)).astype(jnp.bfloat16)
                woh = wo_ref[pl.ds(h, 1), :, :].reshape(DH, D_MODEL)
                oacc[:, :] = oacc[:, :] + jnp.dot(
                    ctx, woh, preferred_element_type=jnp.float32)
                return 0

            lax.fori_loop(0, H, head_body, 0)
            acc[j] = oacc[:, :].astype(jnp.bfloat16)

        def rs_add(s):
            tgt = N_DEV - 2 - s
            acc[tgt] = (acc[tgt].astype(jnp.float32)
                        + rs_recv[s].astype(jnp.float32)
                        ).astype(jnp.bfloat16)

        ag[0].start()
        compute_slot(0)
        ag[0].wait_recv()
        ag[1].start()
        compute_slot(3)
        rs[0].start()
        ag[1].wait_recv()
        ag[2].start()
        compute_slot(2)
        rs[0].wait_recv()
        rs_add(0)
        rs[1].start()
        ag[2].wait_recv()
        compute_slot(1)
        rs[1].wait_recv()
        rs_add(1)
        rs[2].start()
        rs[2].wait_recv()
        rs_add(2)

        out_ref[0] = acc[0].astype(jnp.float32)

        for d in ag:
            d.wait_send()
        for d in rs:
            d.wait_send()

    out_shape = jax.ShapeDtypeStruct((1, SQ, D_MODEL), jnp.float32)
    return pl.pallas_call(
        body,
        out_shape=out_shape,
        in_specs=[pl.BlockSpec(memory_space=pltpu.VMEM)] * 5,
        out_specs=pl.BlockSpec(memory_space=pltpu.VMEM),
        scratch_shapes=[
            pltpu.VMEM((N_DEV, SQ, D_MODEL), jnp.bfloat16),
            pltpu.VMEM((N_DEV, SQ, D_MODEL), jnp.bfloat16),
            pltpu.VMEM((SQ, D_MODEL), jnp.float32),
            pltpu.VMEM((SQ, SKV), jnp.float32),
            pltpu.VMEM((N_DEV - 1, SQ, D_MODEL), jnp.bfloat16),
            pltpu.SemaphoreType.DMA((N_DEV - 1,)),
            pltpu.SemaphoreType.DMA((N_DEV - 1,)),
            pltpu.SemaphoreType.DMA((N_DEV - 1,)),
            pltpu.SemaphoreType.DMA((N_DEV - 1,)),
        ],
        compiler_params=pltpu.CompilerParams(
            collective_id=0, vmem_limit_bytes=100 * 1024 * 1024),
    )(xb, wq_r, k_r, v_r, wo_r)


# device time: 215457 ns/iter; 1.7235x vs baseline; 1.1564x over previous
import jax
import jax.numpy as jnp
from jax import lax
from jax.experimental import pallas as pl
from jax.experimental.pallas import tpu as pltpu

N_DEV = 4
SQ = 1024
SKV = 1024
H = 8
DH = 128
D_MODEL = 1024
SCALE = 0.08838834764831843


def kernel(x, Wq, K_ext, V_ext, Wo):
    i = lax.axis_index("i")

    xb = x.astype(jnp.bfloat16)
    wq_r = Wq.reshape(D_MODEL, H, DH).transpose(1, 0, 2).astype(jnp.bfloat16)
    wo_r = Wo.reshape(H, DH, D_MODEL).astype(jnp.bfloat16)

    k_s = lax.dynamic_slice_in_dim(K_ext, 8 * i, H, axis=2)
    v_s = lax.dynamic_slice_in_dim(V_ext, 8 * i, H, axis=2)
    k_r = k_s.astype(jnp.bfloat16).transpose(0, 2, 1, 3)
    v_r = v_s.astype(jnp.bfloat16).transpose(0, 2, 1, 3)

    def _segments(tq):
        if tq == 0:
            return [(0, SKV // 128)]
        lo, hi = tq - 1, min(tq + 1, SKV // 128 - 1)
        if lo <= 1:
            return [(0, hi + 1)]
        return [(0, 1), (lo, hi + 1)]

    def body(x_ref, wq_ref, k_ref, v_ref, wo_ref, out_ref,
             x_full, acc, oacc, ctx_h, maskadd, rs_recv,
             ag_ss, ag_rs, rs_ss, rs_rs):
        my = lax.axis_index("i")
        left = (my + N_DEV - 1) % N_DEV
        right = (my + 1) % N_DEV

        barrier = pltpu.get_barrier_semaphore()
        pl.semaphore_signal(barrier, inc=1, device_id=(left,),
                            device_id_type=pl.DeviceIdType.MESH)
        pl.semaphore_signal(barrier, inc=1, device_id=(right,),
                            device_id_type=pl.DeviceIdType.MESH)
        pl.semaphore_wait(barrier, 2)

        qi = lax.broadcasted_iota(jnp.int32, (SQ, SKV), 0)
        ki = lax.broadcasted_iota(jnp.int32, (SQ, SKV), 1)
        keep = (jnp.abs(qi - ki) <= 128) | (ki < 32) | (qi < 32)
        maskadd[:, :] = jnp.where(keep, 0.0, -1e9).astype(jnp.float32)

        x_full[0] = x_ref[0]

        ag = [
            pltpu.make_async_remote_copy(
                src_ref=x_full.at[(N_DEV - h) % N_DEV],
                dst_ref=x_full.at[N_DEV - 1 - h],
                send_sem=ag_ss.at[h],
                recv_sem=ag_rs.at[h],
                device_id=(right,),
                device_id_type=pl.DeviceIdType.MESH,
            )
            for h in range(N_DEV - 1)
        ]
        rs = [
            pltpu.make_async_remote_copy(
                src_ref=acc.at[N_DEV - 1 - s],
                dst_ref=rs_recv.at[s],
                send_sem=rs_ss.at[s],
                recv_sem=rs_rs.at[s],
                device_id=(right,),
                device_id_type=pl.DeviceIdType.MESH,
            )
            for s in range(N_DEV - 1)
        ]

        def compute_slot(j):
            x_j = x_full[j]
            b = (my + j) % N_DEV
            oacc[:, :] = jnp.zeros((SQ, D_MODEL), jnp.float32)

            def head_body(h, _):
                wqh = wq_ref[pl.ds(h, 1), :, :].reshape(D_MODEL, DH)
                q = jnp.dot(x_j, wqh, preferred_element_type=jnp.float32)
                q = (q * SCALE).astype(jnp.bfloat16)
                kh = k_ref[pl.ds(b, 1), pl.ds(h, 1), :, :].reshape(SKV, DH)
                vh = v_ref[pl.ds(b, 1), pl.ds(h, 1), :, :].reshape(SKV, DH)
                for tq in range(SQ // 128):
                    r0, r1 = tq * 128, (tq + 1) * 128
                    qt = q[r0:r1, :]
                    ctx_t = None
                    den = None
                    for (c0, c1) in _segments(tq):
                        k0, k1 = c0 * 128, c1 * 128
                        s = lax.dot_general(
                            qt, kh[k0:k1, :], (((1,), (1,)), ((), ())),
                            preferred_element_type=jnp.float32)
                        s = s + maskadd[r0:r1, k0:k1]
                        w = jnp.exp(s)
                        dseg = jnp.sum(w, axis=1, keepdims=True)
                        cseg = jnp.dot(w.astype(jnp.bfloat16), vh[k0:k1, :],
                                       preferred_element_type=jnp.float32)
                        ctx_t = cseg if ctx_t is None else ctx_t + cseg
                        den = dseg if den is None else den + dseg
                    ctx_h[r0:r1, :] = (ctx_t * (1.0 / den)
                                       ).astype(jnp.bfloat16)
                woh = wo_ref[pl.ds(h, 1), :, :].reshape(DH, D_MODEL)
                oacc[:, :] = oacc[:, :] + jnp.dot(
                    ctx_h[:, :], woh, preferred_element_type=jnp.float32)
                return 0

            lax.fori_loop(0, H, head_body, 0)
            acc[j] = oacc[:, :].astype(jnp.bfloat16)

        def rs_add(s):
            tgt = N_DEV - 2 - s
            acc[tgt] = (acc[tgt].astype(jnp.float32)
                        + rs_recv[s].astype(jnp.float32)
                        ).astype(jnp.bfloat16)

        ag[0].start()
        compute_slot(0)
        ag[0].wait_recv()
        ag[1].start()
        compute_slot(3)
        rs[0].start()
        ag[1].wait_recv()
        ag[2].start()
        compute_slot(2)
        rs[0].wait_recv()
        rs_add(0)
        rs[1].start()
        ag[2].wait_recv()
        compute_slot(1)
        rs[1].wait_recv()
        rs_add(1)
        rs[2].start()
        rs[2].wait_recv()
        rs_add(2)

        out_ref[0] = acc[0].astype(jnp.float32)

        for d in ag:
            d.wait_send()
        for d in rs:
            d.wait_send()

    out_shape = jax.ShapeDtypeStruct((1, SQ, D_MODEL), jnp.float32)
    return pl.pallas_call(
        body,
        out_shape=out_shape,
        in_specs=[pl.BlockSpec(memory_space=pltpu.VMEM)] * 5,
        out_specs=pl.BlockSpec(memory_space=pltpu.VMEM),
        scratch_shapes=[
            pltpu.VMEM((N_DEV, SQ, D_MODEL), jnp.bfloat16),
            pltpu.VMEM((N_DEV, SQ, D_MODEL), jnp.bfloat16),
            pltpu.VMEM((SQ, D_MODEL), jnp.float32),
            pltpu.VMEM((SQ, DH), jnp.bfloat16),
            pltpu.VMEM((SQ, SKV), jnp.float32),
            pltpu.VMEM((N_DEV - 1, SQ, D_MODEL), jnp.bfloat16),
            pltpu.SemaphoreType.DMA((N_DEV - 1,)),
            pltpu.SemaphoreType.DMA((N_DEV - 1,)),
            pltpu.SemaphoreType.DMA((N_DEV - 1,)),
            pltpu.SemaphoreType.DMA((N_DEV - 1,)),
        ],
        compiler_params=pltpu.CompilerParams(
            collective_id=0, vmem_limit_bytes=100 * 1024 * 1024),
    )(xb, wq_r, k_r, v_r, wo_r)


# device time: 195393 ns/iter; 1.9005x vs baseline; 1.1027x over previous
import jax
import jax.numpy as jnp
from jax import lax
from jax.experimental import pallas as pl
from jax.experimental.pallas import tpu as pltpu

N_DEV = 4
SQ = 1024
SKV = 1024
H = 8
DH = 128
D_MODEL = 1024
SCALE = 0.08838834764831843


def kernel(x, Wq, K_ext, V_ext, Wo):
    i = lax.axis_index("i")

    xb = x.astype(jnp.bfloat16)
    wq_b = Wq.astype(jnp.bfloat16)
    wo_b = Wo.astype(jnp.bfloat16)

    k_s = lax.dynamic_slice_in_dim(K_ext, 8 * i, H, axis=2)
    v_s = lax.dynamic_slice_in_dim(V_ext, 8 * i, H, axis=2)
    k_r = k_s.astype(jnp.bfloat16).transpose(0, 2, 1, 3)
    v_r = v_s.astype(jnp.bfloat16).transpose(0, 2, 1, 3)

    def _segments(tq):
        if tq == 0:
            return [(0, SKV // 128)]
        lo, hi = tq - 1, min(tq + 1, SKV // 128 - 1)
        if lo <= 1:
            return [(0, hi + 1)]
        return [(0, 1), (lo, hi + 1)]

    def body(x_ref, wq_ref, k_ref, v_ref, wo_ref, out_ref,
             x_full, acc, qall, ctx_all, maskadd, rs_recv,
             ag_ss, ag_rs, rs_ss, rs_rs):
        my = lax.axis_index("i")
        left = (my + N_DEV - 1) % N_DEV
        right = (my + 1) % N_DEV

        barrier = pltpu.get_barrier_semaphore()
        pl.semaphore_signal(barrier, inc=1, device_id=(left,),
                            device_id_type=pl.DeviceIdType.MESH)
        pl.semaphore_signal(barrier, inc=1, device_id=(right,),
                            device_id_type=pl.DeviceIdType.MESH)
        pl.semaphore_wait(barrier, 2)

        qi = lax.broadcasted_iota(jnp.int32, (SQ, SKV), 0)
        ki = lax.broadcasted_iota(jnp.int32, (SQ, SKV), 1)
        keep = (jnp.abs(qi - ki) <= 128) | (ki < 32) | (qi < 32)
        maskadd[:, :] = jnp.where(keep, 0.0, -1e9).astype(jnp.float32)

        x_full[0] = x_ref[0]

        ag = [
            pltpu.make_async_remote_copy(
                src_ref=x_full.at[(N_DEV - h) % N_DEV],
                dst_ref=x_full.at[N_DEV - 1 - h],
                send_sem=ag_ss.at[h],
                recv_sem=ag_rs.at[h],
                device_id=(right,),
                device_id_type=pl.DeviceIdType.MESH,
            )
            for h in range(N_DEV - 1)
        ]
        rs = [
            pltpu.make_async_remote_copy(
                src_ref=acc.at[N_DEV - 1 - s],
                dst_ref=rs_recv.at[s],
                send_sem=rs_ss.at[s],
                recv_sem=rs_rs.at[s],
                device_id=(right,),
                device_id_type=pl.DeviceIdType.MESH,
            )
            for s in range(N_DEV - 1)
        ]

        def compute_slot(j):
            x_j = x_full[j]
            b = (my + j) % N_DEV
            qall[:, :] = (jnp.dot(x_j, wq_ref[:, :],
                                  preferred_element_type=jnp.float32)
                          * SCALE).astype(jnp.bfloat16)

            def head_body(h, _):
                off = pl.multiple_of(h * DH, DH)
                q = qall[:, pl.ds(off, DH)]
                kh = k_ref[pl.ds(b, 1), pl.ds(h, 1), :, :].reshape(SKV, DH)
                vh = v_ref[pl.ds(b, 1), pl.ds(h, 1), :, :].reshape(SKV, DH)
                for tq in range(SQ // 128):
                    r0, r1 = tq * 128, (tq + 1) * 128
                    qt = q[r0:r1, :]
                    ctx_t = None
                    den = None
                    for (c0, c1) in _segments(tq):
                        k0, k1 = c0 * 128, c1 * 128
                        s = lax.dot_general(
                            qt, kh[k0:k1, :], (((1,), (1,)), ((), ())),
                            preferred_element_type=jnp.float32)
                        s = s + maskadd[r0:r1, k0:k1]
                        w = jnp.exp(s)
                        dseg = jnp.sum(w, axis=1, keepdims=True)
                        cseg = jnp.dot(w.astype(jnp.bfloat16), vh[k0:k1, :],
                                       preferred_element_type=jnp.float32)
                        ctx_t = cseg if ctx_t is None else ctx_t + cseg
                        den = dseg if den is None else den + dseg
                    ctx_all[r0:r1, pl.ds(off, DH)] = (
                        ctx_t * (1.0 / den)).astype(jnp.bfloat16)
                return 0

            lax.fori_loop(0, H, head_body, 0)
            acc[j] = jnp.dot(ctx_all[:, :], wo_ref[:, :],
                             preferred_element_type=jnp.float32
                             ).astype(jnp.bfloat16)

        def rs_add(s):
            tgt = N_DEV - 2 - s
            acc[tgt] = (acc[tgt].astype(jnp.float32)
                        + rs_recv[s].astype(jnp.float32)
                        ).astype(jnp.bfloat16)

        ag[0].start()
        compute_slot(0)
        ag[0].wait_recv()
        ag[1].start()
        compute_slot(3)
        rs[0].start()
        ag[1].wait_recv()
        ag[2].start()
        compute_slot(2)
        rs[0].wait_recv()
        rs_add(0)
        rs[1].start()
        ag[2].wait_recv()
        compute_slot(1)
        rs[1].wait_recv()
        rs_add(1)
        rs[2].start()
        rs[2].wait_recv()
        rs_add(2)

        out_ref[0] = acc[0].astype(jnp.float32)

        for d in ag:
            d.wait_send()
        for d in rs:
            d.wait_send()

    out_shape = jax.ShapeDtypeStruct((1, SQ, D_MODEL), jnp.float32)
    return pl.pallas_call(
        body,
        out_shape=out_shape,
        in_specs=[pl.BlockSpec(memory_space=pltpu.VMEM)] * 5,
        out_specs=pl.BlockSpec(memory_space=pltpu.VMEM),
        scratch_shapes=[
            pltpu.VMEM((N_DEV, SQ, D_MODEL), jnp.bfloat16),
            pltpu.VMEM((N_DEV, SQ, D_MODEL), jnp.bfloat16),
            pltpu.VMEM((SQ, H * DH), jnp.bfloat16),
            pltpu.VMEM((SQ, H * DH), jnp.bfloat16),
            pltpu.VMEM((SQ, SKV), jnp.float32),
            pltpu.VMEM((N_DEV - 1, SQ, D_MODEL), jnp.bfloat16),
            pltpu.SemaphoreType.DMA((N_DEV - 1,)),
            pltpu.SemaphoreType.DMA((N_DEV - 1,)),
            pltpu.SemaphoreType.DMA((N_DEV - 1,)),
            pltpu.SemaphoreType.DMA((N_DEV - 1,)),
        ],
        compiler_params=pltpu.CompilerParams(
            collective_id=0, vmem_limit_bytes=100 * 1024 * 1024),
    )(xb, wq_b, k_r, v_r, wo_b)


# device time: 195076 ns/iter; 1.9036x vs baseline; 1.0016x over previous
import jax
import jax.numpy as jnp
from jax import lax
from jax.experimental import pallas as pl
from jax.experimental.pallas import tpu as pltpu

N_DEV = 4
SQ = 1024
SKV = 1024
H = 8
DH = 128
D_MODEL = 1024
SCALE = 0.08838834764831843
LOG2E = 1.4426950408889634
QT = 256
KT = SKV // 128


def kernel(x, Wq, K_ext, V_ext, Wo):
    i = lax.axis_index("i")

    xb = x.astype(jnp.bfloat16)
    wq_b = Wq.astype(jnp.bfloat16)
    wo_b = Wo.astype(jnp.bfloat16)

    k_s = lax.dynamic_slice_in_dim(K_ext, 8 * i, H, axis=2)
    v_s = lax.dynamic_slice_in_dim(V_ext, 8 * i, H, axis=2)
    k_r = k_s.astype(jnp.bfloat16).transpose(0, 2, 1, 3)
    v_r = v_s.astype(jnp.bfloat16).transpose(0, 2, 1, 3)

    def _segments(t):
        if t == 0:
            return [(0, KT)]
        lo = (QT * t - 128) // 128
        hi = min((QT * (t + 1) - 1 + 128) // 128, KT - 1)
        if lo <= 1:
            return [(0, hi + 1)]
        return [(0, 1), (lo, hi + 1)]

    def body(x_ref, wq_ref, k_ref, v_ref, wo_ref, out_ref,
             x_full, acc, qall, ctx_all, maskadd, rs_recv,
             ag_ss, ag_rs, rs_ss, rs_rs):
        my = lax.axis_index("i")
        left = (my + N_DEV - 1) % N_DEV
        right = (my + 1) % N_DEV

        barrier = pltpu.get_barrier_semaphore()
        pl.semaphore_signal(barrier, inc=1, device_id=(left,),
                            device_id_type=pl.DeviceIdType.MESH)
        pl.semaphore_signal(barrier, inc=1, device_id=(right,),
                            device_id_type=pl.DeviceIdType.MESH)
        pl.semaphore_wait(barrier, 2)

        qi = lax.broadcasted_iota(jnp.int32, (SQ, SKV), 0)
        ki = lax.broadcasted_iota(jnp.int32, (SQ, SKV), 1)
        keep = (jnp.abs(qi - ki) <= 128) | (ki < 32) | (qi < 32)
        maskadd[:, :] = jnp.where(keep, 0.0, -1e9).astype(jnp.float32)

        x_full[0] = x_ref[0]

        ag = [
            pltpu.make_async_remote_copy(
                src_ref=x_full.at[(N_DEV - h) % N_DEV],
                dst_ref=x_full.at[N_DEV - 1 - h],
                send_sem=ag_ss.at[h],
                recv_sem=ag_rs.at[h],
                device_id=(right,),
                device_id_type=pl.DeviceIdType.MESH,
            )
            for h in range(N_DEV - 1)
        ]
        rs = [
            pltpu.make_async_remote_copy(
                src_ref=acc.at[N_DEV - 1 - s],
                dst_ref=rs_recv.at[s],
                send_sem=rs_ss.at[s],
                recv_sem=rs_rs.at[s],
                device_id=(right,),
                device_id_type=pl.DeviceIdType.MESH,
            )
            for s in range(N_DEV - 1)
        ]

        def compute_slot(j):
            x_j = x_full[j]
            b = (my + j) % N_DEV
            qall[:, :] = (jnp.dot(x_j, wq_ref[:, :],
                                  preferred_element_type=jnp.float32)
                          * (SCALE * LOG2E)).astype(jnp.bfloat16)

            def head_body(h, _):
                off = pl.multiple_of(h * DH, DH)
                q = qall[:, pl.ds(off, DH)]
                kh = k_ref[pl.ds(b, 1), pl.ds(h, 1), :, :].reshape(SKV, DH)
                vh = v_ref[pl.ds(b, 1), pl.ds(h, 1), :, :].reshape(SKV, DH)
                for tq in range(SQ // QT):
                    r0, r1 = tq * QT, (tq + 1) * QT
                    qt = q[r0:r1, :]
                    ctx_t = None
                    den = None
                    for (c0, c1) in _segments(tq):
                        k0, k1 = c0 * 128, c1 * 128
                        s = lax.dot_general(
                            qt, kh[k0:k1, :], (((1,), (1,)), ((), ())),
                            preferred_element_type=jnp.float32)
                        w = jnp.exp2(s + maskadd[r0:r1, k0:k1])
                        dseg = jnp.sum(w, axis=1, keepdims=True)
                        cseg = jnp.dot(w.astype(jnp.bfloat16), vh[k0:k1, :],
                                       preferred_element_type=jnp.float32)
                        ctx_t = cseg if ctx_t is None else ctx_t + cseg
                        den = dseg if den is None else den + dseg
                    ctx_all[r0:r1, pl.ds(off, DH)] = (
                        ctx_t * (1.0 / den)).astype(jnp.bfloat16)
                return 0

            lax.fori_loop(0, H, head_body, 0)
            acc[j] = jnp.dot(ctx_all[:, :], wo_ref[:, :],
                             preferred_element_type=jnp.float32
                             ).astype(jnp.bfloat16)

        def rs_add(s):
            tgt = N_DEV - 2 - s
            acc[tgt] = (acc[tgt].astype(jnp.float32)
                        + rs_recv[s].astype(jnp.float32)
                        ).astype(jnp.bfloat16)

        ag[0].start()
        compute_slot(0)
        ag[0].wait_recv()
        ag[1].start()
        compute_slot(3)
        rs[0].start()
        ag[1].wait_recv()
        ag[2].start()
        compute_slot(2)
        rs[0].wait_recv()
        rs_add(0)
        rs[1].start()
        ag[2].wait_recv()
        compute_slot(1)
        rs[1].wait_recv()
        rs_add(1)
        rs[2].start()
        rs[2].wait_recv()
        out_ref[0] = (acc[0].astype(jnp.float32)
                      + rs_recv[2].astype(jnp.float32))

        for d in ag:
            d.wait_send()
        for d in rs:
            d.wait_send()

    out_shape = jax.ShapeDtypeStruct((1, SQ, D_MODEL), jnp.float32)
    return pl.pallas_call(
        body,
        out_shape=out_shape,
        in_specs=[pl.BlockSpec(memory_space=pltpu.VMEM)] * 5,
        out_specs=pl.BlockSpec(memory_space=pltpu.VMEM),
        scratch_shapes=[
            pltpu.VMEM((N_DEV, SQ, D_MODEL), jnp.bfloat16),
            pltpu.VMEM((N_DEV, SQ, D_MODEL), jnp.bfloat16),
            pltpu.VMEM((SQ, H * DH), jnp.bfloat16),
            pltpu.VMEM((SQ, H * DH), jnp.bfloat16),
            pltpu.VMEM((SQ, SKV), jnp.float32),
            pltpu.VMEM((N_DEV - 1, SQ, D_MODEL), jnp.bfloat16),
            pltpu.SemaphoreType.DMA((N_DEV - 1,)),
            pltpu.SemaphoreType.DMA((N_DEV - 1,)),
            pltpu.SemaphoreType.DMA((N_DEV - 1,)),
            pltpu.SemaphoreType.DMA((N_DEV - 1,)),
        ],
        compiler_params=pltpu.CompilerParams(
            collective_id=0, vmem_limit_bytes=100 * 1024 * 1024),
    )(xb, wq_b, k_r, v_r, wo_b)


# device time: 194941 ns/iter; 1.9049x vs baseline; 1.0007x over previous
import jax
import jax.numpy as jnp
from jax import lax
from jax.experimental import pallas as pl
from jax.experimental.pallas import tpu as pltpu

N_DEV = 4
SQ = 1024
SKV = 1024
H = 8
DH = 128
D_MODEL = 1024
SCALE = 0.08838834764831843
LOG2E = 1.4426950408889634
QT = 128
KT = SKV // 128


def kernel(x, Wq, K_ext, V_ext, Wo):
    i = lax.axis_index("i")

    xb = x.astype(jnp.bfloat16)
    wq_b = Wq.astype(jnp.bfloat16)
    wo_b = Wo.astype(jnp.bfloat16)

    k_s = lax.dynamic_slice_in_dim(K_ext, 8 * i, H, axis=2)
    v_s = lax.dynamic_slice_in_dim(V_ext, 8 * i, H, axis=2)
    k_r = k_s.astype(jnp.bfloat16).transpose(0, 2, 1, 3)
    v_r = v_s.astype(jnp.bfloat16).transpose(0, 2, 1, 3)

    def _segments(t):
        if t == 0:
            return [(0, KT)]
        lo = (QT * t - 128) // 128
        hi = min((QT * (t + 1) - 1 + 128) // 128, KT - 1)
        if lo <= 1:
            return [(0, hi + 1)]
        return [(0, 1), (lo, hi + 1)]

    def body(x_ref, wq_ref, k_ref, v_ref, wo_ref, out_ref,
             x_full, acc, qall, ctx_all, maskadd, rs_recv,
             ag_ss, ag_rs, rs_ss, rs_rs):
        my = lax.axis_index("i")
        left = (my + N_DEV - 1) % N_DEV
        right = (my + 1) % N_DEV

        barrier = pltpu.get_barrier_semaphore()
        pl.semaphore_signal(barrier, inc=1, device_id=(left,),
                            device_id_type=pl.DeviceIdType.MESH)
        pl.semaphore_signal(barrier, inc=1, device_id=(right,),
                            device_id_type=pl.DeviceIdType.MESH)
        pl.semaphore_wait(barrier, 2)

        qi = lax.broadcasted_iota(jnp.int32, (SQ, SKV), 0)
        ki = lax.broadcasted_iota(jnp.int32, (SQ, SKV), 1)
        keep = (jnp.abs(qi - ki) <= 128) | (ki < 32) | (qi < 32)
        maskadd[:, :] = jnp.where(keep, 0.0, -1e9).astype(jnp.float32)

        x_full[0] = x_ref[0]

        ag = [
            pltpu.make_async_remote_copy(
                src_ref=x_full.at[(N_DEV - h) % N_DEV],
                dst_ref=x_full.at[N_DEV - 1 - h],
                send_sem=ag_ss.at[h],
                recv_sem=ag_rs.at[h],
                device_id=(right,),
                device_id_type=pl.DeviceIdType.MESH,
            )
            for h in range(N_DEV - 1)
        ]
        rs = [
            pltpu.make_async_remote_copy(
                src_ref=acc.at[N_DEV - 1 - s],
                dst_ref=rs_recv.at[s],
                send_sem=rs_ss.at[s],
                recv_sem=rs_rs.at[s],
                device_id=(right,),
                device_id_type=pl.DeviceIdType.MESH,
            )
            for s in range(N_DEV - 1)
        ]

        def compute_slot(j):
            x_j = x_full[j]
            b = (my + j) % N_DEV
            qall[:, :] = (jnp.dot(x_j, wq_ref[:, :],
                                  preferred_element_type=jnp.float32)
                          * (SCALE * LOG2E)).astype(jnp.bfloat16)

            def head_body(h, _):
                off = pl.multiple_of(h * DH, DH)
                q = qall[:, pl.ds(off, DH)]
                kh = k_ref[pl.ds(b, 1), pl.ds(h, 1), :, :].reshape(SKV, DH)
                vh = v_ref[pl.ds(b, 1), pl.ds(h, 1), :, :].reshape(SKV, DH)
                for tq in range(SQ // QT):
                    r0, r1 = tq * QT, (tq + 1) * QT
                    qt = q[r0:r1, :]
                    ctx_t = None
                    den = None
                    for (c0, c1) in _segments(tq):
                        k0, k1 = c0 * 128, c1 * 128
                        s = lax.dot_general(
                            qt, kh[k0:k1, :], (((1,), (1,)), ((), ())),
                            preferred_element_type=jnp.float32)
                        w = jnp.exp2(s + maskadd[r0:r1, k0:k1])
                        dseg = jnp.sum(w, axis=1, keepdims=True)
                        cseg = jnp.dot(w.astype(jnp.bfloat16), vh[k0:k1, :],
                                       preferred_element_type=jnp.float32)
                        ctx_t = cseg if ctx_t is None else ctx_t + cseg
                        den = dseg if den is None else den + dseg
                    ctx_all[r0:r1, pl.ds(off, DH)] = (
                        ctx_t * (1.0 / den)).astype(jnp.bfloat16)
                return 0

            lax.fori_loop(0, H, head_body, 0)
            acc[j] = jnp.dot(ctx_all[:, :], wo_ref[:, :],
                             preferred_element_type=jnp.float32
                             ).astype(jnp.bfloat16)

        def rs_add(s):
            tgt = N_DEV - 2 - s
            acc[tgt] = (acc[tgt].astype(jnp.float32)
                        + rs_recv[s].astype(jnp.float32)
                        ).astype(jnp.bfloat16)

        ag[0].start()
        compute_slot(0)
        ag[0].wait_recv()
        ag[1].start()
        compute_slot(3)
        rs[0].start()
        ag[1].wait_recv()
        ag[2].start()
        compute_slot(2)
        rs[0].wait_recv()
        rs_add(0)
        rs[1].start()
        ag[2].wait_recv()
        compute_slot(1)
        rs[1].wait_recv()
        rs_add(1)
        rs[2].start()
        rs[2].wait_recv()
        out_ref[0] = (acc[0].astype(jnp.float32)
                      + rs_recv[2].astype(jnp.float32))

        for d in ag:
            d.wait_send()
        for d in rs:
            d.wait_send()

    out_shape = jax.ShapeDtypeStruct((1, SQ, D_MODEL), jnp.float32)
    return pl.pallas_call(
        body,
        out_shape=out_shape,
        in_specs=[pl.BlockSpec(memory_space=pltpu.VMEM)] * 5,
        out_specs=pl.BlockSpec(memory_space=pltpu.VMEM),
        scratch_shapes=[
            pltpu.VMEM((N_DEV, SQ, D_MODEL), jnp.bfloat16),
            pltpu.VMEM((N_DEV, SQ, D_MODEL), jnp.bfloat16),
            pltpu.VMEM((SQ, H * DH), jnp.bfloat16),
            pltpu.VMEM((SQ, H * DH), jnp.bfloat16),
            pltpu.VMEM((SQ, SKV), jnp.float32),
            pltpu.VMEM((N_DEV - 1, SQ, D_MODEL), jnp.bfloat16),
            pltpu.SemaphoreType.DMA((N_DEV - 1,)),
            pltpu.SemaphoreType.DMA((N_DEV - 1,)),
            pltpu.SemaphoreType.DMA((N_DEV - 1,)),
            pltpu.SemaphoreType.DMA((N_DEV - 1,)),
        ],
        compiler_params=pltpu.CompilerParams(
            collective_id=0, vmem_limit_bytes=100 * 1024 * 1024),
    )(xb, wq_b, k_r, v_r, wo_b)


# device time: 162775 ns/iter; 2.2813x vs baseline; 1.1976x over previous
import jax
import jax.numpy as jnp
from jax import lax
from jax.experimental import pallas as pl
from jax.experimental.pallas import tpu as pltpu

N_DEV = 4
SQ = 1024
SKV = 1024
H = 8
DH = 128
D_MODEL = 1024
SCALE = 0.08838834764831843
LOG2E = 1.4426950408889634
QT = 128
KT = SKV // 128


def kernel(x, Wq, K_ext, V_ext, Wo):
    i = lax.axis_index("i")

    xb = x.astype(jnp.bfloat16)
    wq_b = Wq.astype(jnp.bfloat16)
    wo_b = Wo.astype(jnp.bfloat16)


    def _segments(t):
        if t == 0:
            return [(0, KT)]
        lo = (QT * t - 128) // 128
        hi = min((QT * (t + 1) - 1 + 128) // 128, KT - 1)
        if lo <= 1:
            return [(0, hi + 1)]
        return [(0, 1), (lo, hi + 1)]

    def body(x_ref, wq_ref, k_hbm, v_hbm, wo_ref, out_ref,
             x_full, acc, qall, ctx_all, maskadd, rs_recv, kbuf, vbuf,
             ag_ss, ag_rs, rs_ss, rs_rs, k_sems, v_sems):
        my = lax.axis_index("i")
        left = (my + N_DEV - 1) % N_DEV
        right = (my + 1) % N_DEV

        barrier = pltpu.get_barrier_semaphore()
        pl.semaphore_signal(barrier, inc=1, device_id=(left,),
                            device_id_type=pl.DeviceIdType.MESH)
        pl.semaphore_signal(barrier, inc=1, device_id=(right,),
                            device_id_type=pl.DeviceIdType.MESH)
        pl.semaphore_wait(barrier, 2)

        qi = lax.broadcasted_iota(jnp.int32, (SQ, SKV), 0)
        ki = lax.broadcasted_iota(jnp.int32, (SQ, SKV), 1)
        keep = (jnp.abs(qi - ki) <= 128) | (ki < 32) | (qi < 32)
        maskadd[:, :] = jnp.where(keep, 0.0, -1e9).astype(jnp.float32)

        x_full[0] = x_ref[0]

        ag = [
            pltpu.make_async_remote_copy(
                src_ref=x_full.at[(N_DEV - h) % N_DEV],
                dst_ref=x_full.at[N_DEV - 1 - h],
                send_sem=ag_ss.at[h],
                recv_sem=ag_rs.at[h],
                device_id=(right,),
                device_id_type=pl.DeviceIdType.MESH,
            )
            for h in range(N_DEV - 1)
        ]
        rs = [
            pltpu.make_async_remote_copy(
                src_ref=acc.at[N_DEV - 1 - s],
                dst_ref=rs_recv.at[s],
                send_sem=rs_ss.at[s],
                recv_sem=rs_rs.at[s],
                device_id=(right,),
                device_id_type=pl.DeviceIdType.MESH,
            )
            for s in range(N_DEV - 1)
        ]

        def kv_copies(j):
            b = (my + j) % N_DEV
            out = []
            for h in range(H):
                hh = H * my + h
                out.append(pltpu.make_async_copy(
                    k_hbm.at[b, :, hh, :], kbuf.at[h], k_sems.at[h]))
                out.append(pltpu.make_async_copy(
                    v_hbm.at[b, :, hh, :], vbuf.at[h], v_sems.at[h]))
            return out

        def compute_slot(j, next_j=None):
            x_j = x_full[j]
            qall[:, :] = (jnp.dot(x_j, wq_ref[:, :],
                                  preferred_element_type=jnp.float32)
                          * (SCALE * LOG2E)).astype(jnp.bfloat16)
            for c in kv_copies(j):
                c.wait()

            def head_body(h, _):
                off = pl.multiple_of(h * DH, DH)
                q = qall[:, pl.ds(off, DH)]
                kh = kbuf[pl.ds(h, 1), :, :].reshape(SKV, DH).astype(
                    jnp.bfloat16)
                vh = vbuf[pl.ds(h, 1), :, :].reshape(SKV, DH).astype(
                    jnp.bfloat16)
                for tq in range(SQ // QT):
                    r0, r1 = tq * QT, (tq + 1) * QT
                    qt = q[r0:r1, :]
                    ctx_t = None
                    den = None
                    for (c0, c1) in _segments(tq):
                        k0, k1 = c0 * 128, c1 * 128
                        s = lax.dot_general(
                            qt, kh[k0:k1, :], (((1,), (1,)), ((), ())),
                            preferred_element_type=jnp.float32)
                        w = jnp.exp2(s + maskadd[r0:r1, k0:k1])
                        dseg = jnp.sum(w, axis=1, keepdims=True)
                        cseg = jnp.dot(w.astype(jnp.bfloat16), vh[k0:k1, :],
                                       preferred_element_type=jnp.float32)
                        ctx_t = cseg if ctx_t is None else ctx_t + cseg
                        den = dseg if den is None else den + dseg
                    ctx_all[r0:r1, pl.ds(off, DH)] = (
                        ctx_t * (1.0 / den)).astype(jnp.bfloat16)
                return 0

            lax.fori_loop(0, H, head_body, 0)
            if next_j is not None:
                for c in kv_copies(next_j):
                    c.start()
            acc[j] = jnp.dot(ctx_all[:, :], wo_ref[:, :],
                             preferred_element_type=jnp.float32
                             ).astype(jnp.bfloat16)

        def rs_add(s):
            tgt = N_DEV - 2 - s
            acc[tgt] = (acc[tgt].astype(jnp.float32)
                        + rs_recv[s].astype(jnp.float32)
                        ).astype(jnp.bfloat16)

        for c in kv_copies(0):
            c.start()
        ag[0].start()
        compute_slot(0, next_j=3)
        ag[0].wait_recv()
        ag[1].start()
        compute_slot(3, next_j=2)
        rs[0].start()
        ag[1].wait_recv()
        ag[2].start()
        compute_slot(2, next_j=1)
        rs[0].wait_recv()
        rs_add(0)
        rs[1].start()
        ag[2].wait_recv()
        compute_slot(1)
        rs[1].wait_recv()
        rs_add(1)
        rs[2].start()
        rs[2].wait_recv()
        out_ref[0] = (acc[0].astype(jnp.float32)
                      + rs_recv[2].astype(jnp.float32))

        for d in ag:
            d.wait_send()
        for d in rs:
            d.wait_send()

    out_shape = jax.ShapeDtypeStruct((1, SQ, D_MODEL), jnp.float32)
    return pl.pallas_call(
        body,
        out_shape=out_shape,
        in_specs=[
            pl.BlockSpec(memory_space=pltpu.VMEM),
            pl.BlockSpec(memory_space=pltpu.VMEM),
            pl.BlockSpec(memory_space=pltpu.MemorySpace.HBM),
            pl.BlockSpec(memory_space=pltpu.MemorySpace.HBM),
            pl.BlockSpec(memory_space=pltpu.VMEM),
        ],
        out_specs=pl.BlockSpec(memory_space=pltpu.VMEM),
        scratch_shapes=[
            pltpu.VMEM((N_DEV, SQ, D_MODEL), jnp.bfloat16),
            pltpu.VMEM((N_DEV, SQ, D_MODEL), jnp.bfloat16),
            pltpu.VMEM((SQ, H * DH), jnp.bfloat16),
            pltpu.VMEM((SQ, H * DH), jnp.bfloat16),
            pltpu.VMEM((SQ, SKV), jnp.float32),
            pltpu.VMEM((N_DEV - 1, SQ, D_MODEL), jnp.bfloat16),
            pltpu.VMEM((H, SKV, DH), jnp.float32),
            pltpu.VMEM((H, SKV, DH), jnp.float32),
            pltpu.SemaphoreType.DMA((N_DEV - 1,)),
            pltpu.SemaphoreType.DMA((N_DEV - 1,)),
            pltpu.SemaphoreType.DMA((N_DEV - 1,)),
            pltpu.SemaphoreType.DMA((N_DEV - 1,)),
            pltpu.SemaphoreType.DMA((H,)),
            pltpu.SemaphoreType.DMA((H,)),
        ],
        compiler_params=pltpu.CompilerParams(
            collective_id=0, vmem_limit_bytes=100 * 1024 * 1024),
    )(xb, wq_b, K_ext, V_ext, wo_b)


# device time: 162239 ns/iter; 2.2889x vs baseline; 1.0033x over previous
import jax
import jax.numpy as jnp
from jax import lax
from jax.experimental import pallas as pl
from jax.experimental.pallas import tpu as pltpu

N_DEV = 4
SQ = 1024
SKV = 1024
H = 8
DH = 128
D_MODEL = 1024
SCALE = 0.08838834764831843
LOG2E = 1.4426950408889634
QT = 128
KT = SKV // 128


def kernel(x, Wq, K_ext, V_ext, Wo):
    i = lax.axis_index("i")

    xb = x.astype(jnp.bfloat16)
    wq_b = Wq.astype(jnp.bfloat16)
    wo_b = Wo.astype(jnp.bfloat16)


    def _segments(t):
        if t == 0:
            return [(0, KT)]
        lo = (QT * t - 128) // 128
        hi = min((QT * (t + 1) - 1 + 128) // 128, KT - 1)
        if lo <= 1:
            return [(0, hi + 1)]
        return [(0, 1), (lo, hi + 1)]

    def body(x_ref, wq_ref, k_hbm, v_hbm, wo_ref, out_ref,
             x_full, acc, qall, ctx_all, maskadd, rs_recv, kbuf, vbuf,
             ag_ss, ag_rs, rs_ss, rs_rs, k_sems, v_sems):
        my = lax.axis_index("i")
        left = (my + N_DEV - 1) % N_DEV
        right = (my + 1) % N_DEV

        barrier = pltpu.get_barrier_semaphore()
        pl.semaphore_signal(barrier, inc=1, device_id=(left,),
                            device_id_type=pl.DeviceIdType.MESH)
        pl.semaphore_signal(barrier, inc=1, device_id=(right,),
                            device_id_type=pl.DeviceIdType.MESH)
        pl.semaphore_wait(barrier, 2)

        ag = [
            pltpu.make_async_remote_copy(
                src_ref=(x_ref.at[0] if h == 0
                         else x_full.at[(N_DEV - h) % N_DEV]),
                dst_ref=x_full.at[N_DEV - 1 - h],
                send_sem=ag_ss.at[h],
                recv_sem=ag_rs.at[h],
                device_id=(right,),
                device_id_type=pl.DeviceIdType.MESH,
            )
            for h in range(N_DEV - 1)
        ]
        rs = [
            pltpu.make_async_remote_copy(
                src_ref=acc.at[N_DEV - 1 - s],
                dst_ref=rs_recv.at[s],
                send_sem=rs_ss.at[s],
                recv_sem=rs_rs.at[s],
                device_id=(right,),
                device_id_type=pl.DeviceIdType.MESH,
            )
            for s in range(N_DEV - 1)
        ]

        def kv_copies(j):
            b = (my + j) % N_DEV
            out = []
            for h in range(H):
                hh = H * my + h
                out.append(pltpu.make_async_copy(
                    k_hbm.at[b, :, hh, :], kbuf.at[h], k_sems.at[h]))
                out.append(pltpu.make_async_copy(
                    v_hbm.at[b, :, hh, :], vbuf.at[h], v_sems.at[h]))
            return out

        def compute_slot(j, next_j=None):
            x_j = x_ref[0] if j == 0 else x_full[j]
            qall[:, :] = (jnp.dot(x_j, wq_ref[:, :],
                                  preferred_element_type=jnp.float32)
                          * (SCALE * LOG2E)).astype(jnp.bfloat16)
            for c in kv_copies(j):
                c.wait()

            def head_body(h, _):
                off = pl.multiple_of(h * DH, DH)
                q = qall[:, pl.ds(off, DH)]
                kh = kbuf[pl.ds(h, 1), :, :].reshape(SKV, DH).astype(
                    jnp.bfloat16)
                vh = vbuf[pl.ds(h, 1), :, :].reshape(SKV, DH).astype(
                    jnp.bfloat16)
                for tq in range(SQ // QT):
                    r0, r1 = tq * QT, (tq + 1) * QT
                    qt = q[r0:r1, :]
                    ctx_t = None
                    den = None
                    for (c0, c1) in _segments(tq):
                        k0, k1 = c0 * 128, c1 * 128
                        s = lax.dot_general(
                            qt, kh[k0:k1, :], (((1,), (1,)), ((), ())),
                            preferred_element_type=jnp.float32)
                        w = jnp.exp2(s + maskadd[r0:r1, k0:k1])
                        dseg = jnp.sum(w, axis=1, keepdims=True)
                        cseg = jnp.dot(w.astype(jnp.bfloat16), vh[k0:k1, :],
                                       preferred_element_type=jnp.float32)
                        ctx_t = cseg if ctx_t is None else ctx_t + cseg
                        den = dseg if den is None else den + dseg
                    ctx_all[r0:r1, pl.ds(off, DH)] = (
                        ctx_t * (1.0 / den)).astype(jnp.bfloat16)
                return 0

            lax.fori_loop(0, H, head_body, 0)
            if next_j is not None:
                for c in kv_copies(next_j):
                    c.start()
            acc[j] = jnp.dot(ctx_all[:, :], wo_ref[:, :],
                             preferred_element_type=jnp.float32
                             ).astype(jnp.bfloat16)

        def rs_add(s):
            tgt = N_DEV - 2 - s
            acc[tgt] = (acc[tgt].astype(jnp.float32)
                        + rs_recv[s].astype(jnp.float32)
                        ).astype(jnp.bfloat16)

        for c in kv_copies(0):
            c.start()
        ag[0].start()
        qi = lax.broadcasted_iota(jnp.int32, (SQ, SKV), 0)
        ki = lax.broadcasted_iota(jnp.int32, (SQ, SKV), 1)
        keep = (jnp.abs(qi - ki) <= 128) | (ki < 32) | (qi < 32)
        maskadd[:, :] = jnp.where(keep, 0.0, -1e9).astype(jnp.float32)

        compute_slot(0, next_j=3)
        ag[0].wait_recv()
        ag[1].start()
        compute_slot(3, next_j=2)
        rs[0].start()
        ag[1].wait_recv()
        ag[2].start()
        compute_slot(2, next_j=1)
        rs[0].wait_recv()
        rs_add(0)
        rs[1].start()
        ag[2].wait_recv()
        compute_slot(1)
        rs[1].wait_recv()
        rs_add(1)
        rs[2].start()
        rs[2].wait_recv()
        out_ref[0] = (acc[0].astype(jnp.float32)
                      + rs_recv[2].astype(jnp.float32))

        for d in ag:
            d.wait_send()
        for d in rs:
            d.wait_send()

    out_shape = jax.ShapeDtypeStruct((1, SQ, D_MODEL), jnp.float32)
    return pl.pallas_call(
        body,
        out_shape=out_shape,
        in_specs=[
            pl.BlockSpec(memory_space=pltpu.VMEM),
            pl.BlockSpec(memory_space=pltpu.VMEM),
            pl.BlockSpec(memory_space=pltpu.MemorySpace.HBM),
            pl.BlockSpec(memory_space=pltpu.MemorySpace.HBM),
            pl.BlockSpec(memory_space=pltpu.VMEM),
        ],
        out_specs=pl.BlockSpec(memory_space=pltpu.VMEM),
        scratch_shapes=[
            pltpu.VMEM((N_DEV, SQ, D_MODEL), jnp.bfloat16),
            pltpu.VMEM((N_DEV, SQ, D_MODEL), jnp.bfloat16),
            pltpu.VMEM((SQ, H * DH), jnp.bfloat16),
            pltpu.VMEM((SQ, H * DH), jnp.bfloat16),
            pltpu.VMEM((SQ, SKV), jnp.float32),
            pltpu.VMEM((N_DEV - 1, SQ, D_MODEL), jnp.bfloat16),
            pltpu.VMEM((H, SKV, DH), jnp.float32),
            pltpu.VMEM((H, SKV, DH), jnp.float32),
            pltpu.SemaphoreType.DMA((N_DEV - 1,)),
            pltpu.SemaphoreType.DMA((N_DEV - 1,)),
            pltpu.SemaphoreType.DMA((N_DEV - 1,)),
            pltpu.SemaphoreType.DMA((N_DEV - 1,)),
            pltpu.SemaphoreType.DMA((H,)),
            pltpu.SemaphoreType.DMA((H,)),
        ],
        compiler_params=pltpu.CompilerParams(
            collective_id=0, vmem_limit_bytes=100 * 1024 * 1024),
    )(xb, wq_b, K_ext, V_ext, wo_b)


# device time: 141506 ns/iter; 2.6242x vs baseline; 1.1465x over previous
import jax
import jax.numpy as jnp
from jax import lax
from jax.experimental import pallas as pl
from jax.experimental.pallas import tpu as pltpu

N_DEV = 4
SQ = 1024
SKV = 1024
H = 8
DH = 128
D_MODEL = 1024
SCALE = 0.08838834764831843
LOG2E = 1.4426950408889634
QT = 128
KT = SKV // 128


def kernel(x, Wq, K_ext, V_ext, Wo):
    i = lax.axis_index("i")

    xb = x.astype(jnp.bfloat16)
    wq_b = Wq.astype(jnp.bfloat16)
    wo_b = Wo.astype(jnp.bfloat16)


    def _segments(t):
        if t == 0:
            return [(0, KT)]
        lo = (QT * t - 128) // 128
        hi = min((QT * (t + 1) - 1 + 128) // 128, KT - 1)
        if lo <= 1:
            return [(0, hi + 1)]
        return [(0, 1), (lo, hi + 1)]

    def body(x_ref, wq_ref, k_hbm, v_hbm, wo_ref, out_ref,
             x_full, acc, qall, ctx_all, maskadd, rs_recv, kbuf, vbuf,
             ag_ss, ag_rs, al_ss, al_rs, rs_ss, rs_rs, k_sems, v_sems):
        my = lax.axis_index("i")
        left = (my + N_DEV - 1) % N_DEV
        right = (my + 1) % N_DEV

        barrier = pltpu.get_barrier_semaphore()
        pl.semaphore_signal(barrier, inc=1, device_id=(left,),
                            device_id_type=pl.DeviceIdType.MESH)
        pl.semaphore_signal(barrier, inc=1, device_id=(right,),
                            device_id_type=pl.DeviceIdType.MESH)
        pl.semaphore_wait(barrier, 2)

        ag = [
            pltpu.make_async_remote_copy(
                src_ref=x_ref.at[0],
                dst_ref=x_full.at[3],
                send_sem=ag_ss.at[0],
                recv_sem=ag_rs.at[0],
                device_id=(right,),
                device_id_type=pl.DeviceIdType.MESH,
            ),
            pltpu.make_async_remote_copy(
                src_ref=x_full.at[3],
                dst_ref=x_full.at[2],
                send_sem=ag_ss.at[1],
                recv_sem=ag_rs.at[1],
                device_id=(right,),
                device_id_type=pl.DeviceIdType.MESH,
            ),
        ]
        ag_l = pltpu.make_async_remote_copy(
            src_ref=x_ref.at[0],
            dst_ref=x_full.at[1],
            send_sem=al_ss.at[0],
            recv_sem=al_rs.at[0],
            device_id=(left,),
            device_id_type=pl.DeviceIdType.MESH,
        )
        rs = [
            pltpu.make_async_remote_copy(
                src_ref=acc.at[N_DEV - 1 - s],
                dst_ref=rs_recv.at[s],
                send_sem=rs_ss.at[s],
                recv_sem=rs_rs.at[s],
                device_id=(right,),
                device_id_type=pl.DeviceIdType.MESH,
            )
            for s in range(N_DEV - 1)
        ]

        def kv_copies(j):
            b = (my + j) % N_DEV
            out = []
            for h in range(H):
                hh = H * my + h
                out.append(pltpu.make_async_copy(
                    k_hbm.at[b, :, hh, :], kbuf.at[h], k_sems.at[h]))
                out.append(pltpu.make_async_copy(
                    v_hbm.at[b, :, hh, :], vbuf.at[h], v_sems.at[h]))
            return out

        def compute_slot(j, next_j=None):
            x_j = x_ref[0] if j == 0 else x_full[j]
            qall[:, :] = (jnp.dot(x_j, wq_ref[:, :],
                                  preferred_element_type=jnp.float32)
                          * (SCALE * LOG2E)).astype(jnp.bfloat16)
            for c in kv_copies(j):
                c.wait()

            def head_body(h, _):
                off = pl.multiple_of(h * DH, DH)
                q = qall[:, pl.ds(off, DH)]
                kh = kbuf[pl.ds(h, 1), :, :].reshape(SKV, DH).astype(
                    jnp.bfloat16)
                vh = vbuf[pl.ds(h, 1), :, :].reshape(SKV, DH).astype(
                    jnp.bfloat16)
                for tq in range(SQ // QT):
                    r0, r1 = tq * QT, (tq + 1) * QT
                    qt = q[r0:r1, :]
                    ctx_t = None
                    den = None
                    for (c0, c1) in _segments(tq):
                        k0, k1 = c0 * 128, c1 * 128
                        s = lax.dot_general(
                            qt, kh[k0:k1, :], (((1,), (1,)), ((), ())),
                            preferred_element_type=jnp.float32)
                        w = jnp.exp2(s + maskadd[r0:r1, k0:k1])
                        dseg = jnp.sum(w, axis=1, keepdims=True)
                        cseg = jnp.dot(w.astype(jnp.bfloat16), vh[k0:k1, :],
                                       preferred_element_type=jnp.float32)
                        ctx_t = cseg if ctx_t is None else ctx_t + cseg
                        den = dseg if den is None else den + dseg
                    ctx_all[r0:r1, pl.ds(off, DH)] = (
                        ctx_t * (1.0 / den)).astype(jnp.bfloat16)
                return 0

            lax.fori_loop(0, H, head_body, 0)
            if next_j is not None:
                for c in kv_copies(next_j):
                    c.start()
            acc[j] = jnp.dot(ctx_all[:, :], wo_ref[:, :],
                             preferred_element_type=jnp.float32
                             ).astype(jnp.bfloat16)

        def rs_add(s):
            tgt = N_DEV - 2 - s
            acc[tgt] = (acc[tgt].astype(jnp.float32)
                        + rs_recv[s].astype(jnp.float32)
                        ).astype(jnp.bfloat16)

        for c in kv_copies(0):
            c.start()
        ag[0].start()
        ag_l.start()
        qi = lax.broadcasted_iota(jnp.int32, (SQ, SKV), 0)
        ki = lax.broadcasted_iota(jnp.int32, (SQ, SKV), 1)
        keep = (jnp.abs(qi - ki) <= 128) | (ki < 32) | (qi < 32)
        maskadd[:, :] = jnp.where(keep, 0.0, -1e9).astype(jnp.float32)

        compute_slot(0, next_j=3)
        ag[0].wait_recv()
        ag[1].start()
        compute_slot(3, next_j=2)
        rs[0].start()
        ag[1].wait_recv()
        compute_slot(2, next_j=1)
        rs[0].wait_recv()
        rs_add(0)
        rs[1].start()
        ag_l.wait_recv()
        compute_slot(1)
        rs[1].wait_recv()
        rs_add(1)
        rs[2].start()
        rs[2].wait_recv()
        out_ref[0] = (acc[0].astype(jnp.float32)
                      + rs_recv[2].astype(jnp.float32))

        for d in ag:
            d.wait_send()
        ag_l.wait_send()
        for d in rs:
            d.wait_send()

    out_shape = jax.ShapeDtypeStruct((1, SQ, D_MODEL), jnp.float32)
    return pl.pallas_call(
        body,
        out_shape=out_shape,
        in_specs=[
            pl.BlockSpec(memory_space=pltpu.VMEM),
            pl.BlockSpec(memory_space=pltpu.VMEM),
            pl.BlockSpec(memory_space=pltpu.MemorySpace.HBM),
            pl.BlockSpec(memory_space=pltpu.MemorySpace.HBM),
            pl.BlockSpec(memory_space=pltpu.VMEM),
        ],
        out_specs=pl.BlockSpec(memory_space=pltpu.VMEM),
        scratch_shapes=[
            pltpu.VMEM((N_DEV, SQ, D_MODEL), jnp.bfloat16),
            pltpu.VMEM((N_DEV, SQ, D_MODEL), jnp.bfloat16),
            pltpu.VMEM((SQ, H * DH), jnp.bfloat16),
            pltpu.VMEM((SQ, H * DH), jnp.bfloat16),
            pltpu.VMEM((SQ, SKV), jnp.float32),
            pltpu.VMEM((N_DEV - 1, SQ, D_MODEL), jnp.bfloat16),
            pltpu.VMEM((H, SKV, DH), jnp.float32),
            pltpu.VMEM((H, SKV, DH), jnp.float32),
            pltpu.SemaphoreType.DMA((2,)),
            pltpu.SemaphoreType.DMA((2,)),
            pltpu.SemaphoreType.DMA((1,)),
            pltpu.SemaphoreType.DMA((1,)),
            pltpu.SemaphoreType.DMA((N_DEV - 1,)),
            pltpu.SemaphoreType.DMA((N_DEV - 1,)),
            pltpu.SemaphoreType.DMA((H,)),
            pltpu.SemaphoreType.DMA((H,)),
        ],
        compiler_params=pltpu.CompilerParams(
            collective_id=0, vmem_limit_bytes=100 * 1024 * 1024),
    )(xb, wq_b, K_ext, V_ext, wo_b)


# device time: 141492 ns/iter; 2.6245x vs baseline; 1.0001x over previous
import jax
import jax.numpy as jnp
from jax import lax
from jax.experimental import pallas as pl
from jax.experimental.pallas import tpu as pltpu

N_DEV = 4
SQ = 1024
SKV = 1024
H = 8
DH = 128
D_MODEL = 1024
SCALE = 0.08838834764831843
LOG2E = 1.4426950408889634
QT = 128
KT = SKV // 128


def kernel(x, Wq, K_ext, V_ext, Wo):
    i = lax.axis_index("i")

    xb = x.astype(jnp.bfloat16)
    wq_b = (Wq * (SCALE * LOG2E)).astype(jnp.bfloat16)
    wo_b = Wo.astype(jnp.bfloat16)


    def _segments(t):
        if t == 0:
            return [(0, KT)]
        lo = (QT * t - 128) // 128
        hi = min((QT * (t + 1) - 1 + 128) // 128, KT - 1)
        if lo <= 1:
            return [(0, hi + 1)]
        return [(0, 1), (lo, hi + 1)]

    def body(x_ref, wq_ref, k_hbm, v_hbm, wo_ref, out_ref,
             x_full, acc, qall, q0buf, ctx_all, maskadd, rs_recv, kbuf, vbuf,
             ag_ss, ag_rs, al_ss, al_rs, rs_ss, rs_rs, k_sems, v_sems):
        my = lax.axis_index("i")
        left = (my + N_DEV - 1) % N_DEV
        right = (my + 1) % N_DEV

        barrier = pltpu.get_barrier_semaphore()
        pl.semaphore_signal(barrier, inc=1, device_id=(left,),
                            device_id_type=pl.DeviceIdType.MESH)
        pl.semaphore_signal(barrier, inc=1, device_id=(right,),
                            device_id_type=pl.DeviceIdType.MESH)
        pl.semaphore_wait(barrier, 2)

        ag = [
            pltpu.make_async_remote_copy(
                src_ref=x_ref.at[0],
                dst_ref=x_full.at[3],
                send_sem=ag_ss.at[0],
                recv_sem=ag_rs.at[0],
                device_id=(right,),
                device_id_type=pl.DeviceIdType.MESH,
            ),
            pltpu.make_async_remote_copy(
                src_ref=x_full.at[3],
                dst_ref=x_full.at[2],
                send_sem=ag_ss.at[1],
                recv_sem=ag_rs.at[1],
                device_id=(right,),
                device_id_type=pl.DeviceIdType.MESH,
            ),
        ]
        ag_l = pltpu.make_async_remote_copy(
            src_ref=x_ref.at[0],
            dst_ref=x_full.at[1],
            send_sem=al_ss.at[0],
            recv_sem=al_rs.at[0],
            device_id=(left,),
            device_id_type=pl.DeviceIdType.MESH,
        )
        rs = [
            pltpu.make_async_remote_copy(
                src_ref=acc.at[N_DEV - 1 - s],
                dst_ref=rs_recv.at[s],
                send_sem=rs_ss.at[s],
                recv_sem=rs_rs.at[s],
                device_id=(right,),
                device_id_type=pl.DeviceIdType.MESH,
            )
            for s in range(N_DEV - 1)
        ]

        def kv_copies(j):
            b = (my + j) % N_DEV
            out = []
            for h in range(H):
                hh = H * my + h
                out.append(pltpu.make_async_copy(
                    k_hbm.at[b, :, hh, :], kbuf.at[h], k_sems.at[h]))
                out.append(pltpu.make_async_copy(
                    v_hbm.at[b, :, hh, :], vbuf.at[h], v_sems.at[h]))
            return out

        def proj_q(j, qdst):
            x_j = x_ref[0] if j == 0 else x_full[j]
            qdst[:, :] = jnp.dot(x_j, wq_ref[:, :],
                                 preferred_element_type=jnp.float32
                                 ).astype(jnp.bfloat16)

        def attn_out(j, qsrc, next_j=None):
            for c in kv_copies(j):
                c.wait()

            def head_body(h, _):
                off = pl.multiple_of(h * DH, DH)
                q = qsrc[:, pl.ds(off, DH)]
                kh = kbuf[pl.ds(h, 1), :, :].reshape(SKV, DH).astype(
                    jnp.bfloat16)
                vh = vbuf[pl.ds(h, 1), :, :].reshape(SKV, DH).astype(
                    jnp.bfloat16)
                for tq in range(SQ // QT):
                    r0, r1 = tq * QT, (tq + 1) * QT
                    qt = q[r0:r1, :]
                    ctx_t = None
                    den = None
                    for (c0, c1) in _segments(tq):
                        k0, k1 = c0 * 128, c1 * 128
                        s = lax.dot_general(
                            qt, kh[k0:k1, :], (((1,), (1,)), ((), ())),
                            preferred_element_type=jnp.float32)
                        w = jnp.exp2(s + maskadd[r0:r1, k0:k1])
                        dseg = jnp.sum(w, axis=1, keepdims=True)
                        cseg = jnp.dot(w.astype(jnp.bfloat16), vh[k0:k1, :],
                                       preferred_element_type=jnp.float32)
                        ctx_t = cseg if ctx_t is None else ctx_t + cseg
                        den = dseg if den is None else den + dseg
                    ctx_all[r0:r1, pl.ds(off, DH)] = (
                        ctx_t * (1.0 / den)).astype(jnp.bfloat16)
                return 0

            lax.fori_loop(0, H, head_body, 0)
            if next_j is not None:
                for c in kv_copies(next_j):
                    c.start()
            acc[j] = jnp.dot(ctx_all[:, :], wo_ref[:, :],
                             preferred_element_type=jnp.float32
                             ).astype(jnp.bfloat16)

        def rs_add(s):
            tgt = N_DEV - 2 - s
            acc[tgt] = (acc[tgt].astype(jnp.float32)
                        + rs_recv[s].astype(jnp.float32)
                        ).astype(jnp.bfloat16)

        def compute_slot(j, next_j=None):
            proj_q(j, qall)
            attn_out(j, qall, next_j)

        for c in kv_copies(3):
            c.start()
        ag[0].start()
        ag_l.start()
        qi = lax.broadcasted_iota(jnp.int32, (SQ, SKV), 0)
        ki = lax.broadcasted_iota(jnp.int32, (SQ, SKV), 1)
        keep = (jnp.abs(qi - ki) <= 128) | (ki < 32) | (qi < 32)
        maskadd[:, :] = jnp.where(keep, 0.0, -1e9).astype(jnp.float32)

        proj_q(0, q0buf)
        ag[0].wait_recv()
        ag[1].start()
        compute_slot(3, next_j=2)
        rs[0].start()
        ag[1].wait_recv()
        compute_slot(2, next_j=1)
        rs[0].wait_recv()
        rs_add(0)
        rs[1].start()
        ag_l.wait_recv()
        compute_slot(1, next_j=0)
        rs[1].wait_recv()
        rs_add(1)
        rs[2].start()
        attn_out(0, q0buf)
        rs[2].wait_recv()
        out_ref[0] = (acc[0].astype(jnp.float32)
                      + rs_recv[2].astype(jnp.float32))

        for d in ag:
            d.wait_send()
        ag_l.wait_send()
        for d in rs:
            d.wait_send()

    out_shape = jax.ShapeDtypeStruct((1, SQ, D_MODEL), jnp.float32)
    return pl.pallas_call(
        body,
        out_shape=out_shape,
        in_specs=[
            pl.BlockSpec(memory_space=pltpu.VMEM),
            pl.BlockSpec(memory_space=pltpu.VMEM),
            pl.BlockSpec(memory_space=pltpu.MemorySpace.HBM),
            pl.BlockSpec(memory_space=pltpu.MemorySpace.HBM),
            pl.BlockSpec(memory_space=pltpu.VMEM),
        ],
        out_specs=pl.BlockSpec(memory_space=pltpu.VMEM),
        scratch_shapes=[
            pltpu.VMEM((N_DEV, SQ, D_MODEL), jnp.bfloat16),
            pltpu.VMEM((N_DEV, SQ, D_MODEL), jnp.bfloat16),
            pltpu.VMEM((SQ, H * DH), jnp.bfloat16),
            pltpu.VMEM((SQ, H * DH), jnp.bfloat16),
            pltpu.VMEM((SQ, H * DH), jnp.bfloat16),
            pltpu.VMEM((SQ, SKV), jnp.float32),
            pltpu.VMEM((N_DEV - 1, SQ, D_MODEL), jnp.bfloat16),
            pltpu.VMEM((H, SKV, DH), jnp.float32),
            pltpu.VMEM((H, SKV, DH), jnp.float32),
            pltpu.SemaphoreType.DMA((2,)),
            pltpu.SemaphoreType.DMA((2,)),
            pltpu.SemaphoreType.DMA((1,)),
            pltpu.SemaphoreType.DMA((1,)),
            pltpu.SemaphoreType.DMA((N_DEV - 1,)),
            pltpu.SemaphoreType.DMA((N_DEV - 1,)),
            pltpu.SemaphoreType.DMA((H,)),
            pltpu.SemaphoreType.DMA((H,)),
        ],
        compiler_params=pltpu.CompilerParams(
            collective_id=0, vmem_limit_bytes=100 * 1024 * 1024),
    )(xb, wq_b, K_ext, V_ext, wo_b)


# device time: 138751 ns/iter; 2.6763x vs baseline; 1.0198x over previous
import jax
import jax.numpy as jnp
from jax import lax
from jax.experimental import pallas as pl
from jax.experimental.pallas import tpu as pltpu

N_DEV = 4
SQ = 1024
SKV = 1024
H = 8
DH = 128
D_MODEL = 1024
SCALE = 0.08838834764831843
LOG2E = 1.4426950408889634
QT = 128
KT = SKV // 128


def kernel(x, Wq, K_ext, V_ext, Wo):
    i = lax.axis_index("i")

    xb = x.astype(jnp.bfloat16)
    wq_b = (Wq * (SCALE * LOG2E)).astype(jnp.bfloat16)
    wo_b = Wo.astype(jnp.bfloat16)


    def _segments(t):
        if t == 0:
            return [(0, KT)]
        lo = (QT * t - 128) // 128
        hi = min((QT * (t + 1) - 1 + 128) // 128, KT - 1)
        if lo <= 1:
            return [(0, hi + 1)]
        return [(0, 1), (lo, hi + 1)]

    def body(x_ref, wq_ref, k_hbm, v_hbm, wo_ref, out_ref,
             x_full, acc, qall, q0buf, ctx_all, maskadd, rs_recv, kbuf, vbuf,
             ag_ss, ag_rs, al_ss, al_rs, rs_ss, rs_rs, k_sems, v_sems):
        my = lax.axis_index("i")
        left = (my + N_DEV - 1) % N_DEV
        right = (my + 1) % N_DEV

        barrier = pltpu.get_barrier_semaphore()
        pl.semaphore_signal(barrier, inc=1, device_id=(left,),
                            device_id_type=pl.DeviceIdType.MESH)
        pl.semaphore_signal(barrier, inc=1, device_id=(right,),
                            device_id_type=pl.DeviceIdType.MESH)
        pl.semaphore_wait(barrier, 2)

        ag = [
            pltpu.make_async_remote_copy(
                src_ref=x_ref.at[0],
                dst_ref=x_full.at[3],
                send_sem=ag_ss.at[0],
                recv_sem=ag_rs.at[0],
                device_id=(right,),
                device_id_type=pl.DeviceIdType.MESH,
            ),
            pltpu.make_async_remote_copy(
                src_ref=x_full.at[1],
                dst_ref=x_full.at[2],
                send_sem=ag_ss.at[1],
                recv_sem=ag_rs.at[1],
                device_id=(left,),
                device_id_type=pl.DeviceIdType.MESH,
            ),
        ]
        ag_l = pltpu.make_async_remote_copy(
            src_ref=x_ref.at[0],
            dst_ref=x_full.at[1],
            send_sem=al_ss.at[0],
            recv_sem=al_rs.at[0],
            device_id=(left,),
            device_id_type=pl.DeviceIdType.MESH,
        )
        rs_c = pltpu.make_async_remote_copy(
            src_ref=acc.at[3], dst_ref=rs_recv.at[0],
            send_sem=rs_ss.at[0], recv_sem=rs_rs.at[0],
            device_id=(left,), device_id_type=pl.DeviceIdType.MESH,
        )
        rs_b = pltpu.make_async_remote_copy(
            src_ref=acc.at[2], dst_ref=rs_recv.at[1],
            send_sem=rs_ss.at[1], recv_sem=rs_rs.at[1],
            device_id=(right,), device_id_type=pl.DeviceIdType.MESH,
        )
        rs_a = pltpu.make_async_remote_copy(
            src_ref=acc.at[1], dst_ref=rs_recv.at[2],
            send_sem=rs_ss.at[2], recv_sem=rs_rs.at[2],
            device_id=(right,), device_id_type=pl.DeviceIdType.MESH,
        )

        def kv_copies(j):
            b = (my + j) % N_DEV
            out = []
            for h in range(H):
                hh = H * my + h
                out.append(pltpu.make_async_copy(
                    k_hbm.at[b, :, hh, :], kbuf.at[h], k_sems.at[h]))
                out.append(pltpu.make_async_copy(
                    v_hbm.at[b, :, hh, :], vbuf.at[h], v_sems.at[h]))
            return out

        def proj_q(j, qdst):
            x_j = x_ref[0] if j == 0 else x_full[j]
            qdst[:, :] = jnp.dot(x_j, wq_ref[:, :],
                                 preferred_element_type=jnp.float32
                                 ).astype(jnp.bfloat16)

        def attn_out(j, qsrc, next_j=None):
            for c in kv_copies(j):
                c.wait()

            def head_body(h, _):
                off = pl.multiple_of(h * DH, DH)
                q = qsrc[:, pl.ds(off, DH)]
                kh = kbuf[pl.ds(h, 1), :, :].reshape(SKV, DH).astype(
                    jnp.bfloat16)
                vh = vbuf[pl.ds(h, 1), :, :].reshape(SKV, DH).astype(
                    jnp.bfloat16)
                for tq in range(SQ // QT):
                    r0, r1 = tq * QT, (tq + 1) * QT
                    qt = q[r0:r1, :]
                    ctx_t = None
                    den = None
                    for (c0, c1) in _segments(tq):
                        k0, k1 = c0 * 128, c1 * 128
                        s = lax.dot_general(
                            qt, kh[k0:k1, :], (((1,), (1,)), ((), ())),
                            preferred_element_type=jnp.float32)
                        w = jnp.exp2(s + maskadd[r0:r1, k0:k1])
                        dseg = jnp.sum(w, axis=1, keepdims=True)
                        cseg = jnp.dot(w.astype(jnp.bfloat16), vh[k0:k1, :],
                                       preferred_element_type=jnp.float32)
                        ctx_t = cseg if ctx_t is None else ctx_t + cseg
                        den = dseg if den is None else den + dseg
                    ctx_all[r0:r1, pl.ds(off, DH)] = (
                        ctx_t * (1.0 / den)).astype(jnp.bfloat16)
                return 0

            lax.fori_loop(0, H, head_body, 0)
            if next_j is not None:
                for c in kv_copies(next_j):
                    c.start()
            acc[j] = jnp.dot(ctx_all[:, :], wo_ref[:, :],
                             preferred_element_type=jnp.float32
                             ).astype(jnp.bfloat16)

        def rs_add(s):
            tgt = N_DEV - 2 - s
            acc[tgt] = (acc[tgt].astype(jnp.float32)
                        + rs_recv[s].astype(jnp.float32)
                        ).astype(jnp.bfloat16)

        def compute_slot(j, next_j=None):
            proj_q(j, qall)
            attn_out(j, qall, next_j)

        for c in kv_copies(3):
            c.start()
        ag[0].start()
        ag_l.start()
        qi = lax.broadcasted_iota(jnp.int32, (SQ, SKV), 0)
        ki = lax.broadcasted_iota(jnp.int32, (SQ, SKV), 1)
        keep = (jnp.abs(qi - ki) <= 128) | (ki < 32) | (qi < 32)
        maskadd[:, :] = jnp.where(keep, 0.0, -1e9).astype(jnp.float32)

        proj_q(0, q0buf)
        ag_l.wait_recv()
        ag[1].start()
        ag[0].wait_recv()
        compute_slot(3, next_j=2)
        rs_c.start()
        ag[1].wait_recv()
        compute_slot(2, next_j=1)
        rs_b.start()
        compute_slot(1, next_j=0)
        rs_b.wait_recv()
        acc[1] = (acc[1].astype(jnp.float32)
                  + rs_recv[1].astype(jnp.float32)).astype(jnp.bfloat16)
        rs_a.start()
        attn_out(0, q0buf)
        rs_c.wait_recv()
        rs_a.wait_recv()
        out_ref[0] = (acc[0].astype(jnp.float32)
                      + rs_recv[0].astype(jnp.float32)
                      + rs_recv[2].astype(jnp.float32))

        for d in ag:
            d.wait_send()
        ag_l.wait_send()
        for d in (rs_c, rs_b, rs_a):
            d.wait_send()

    out_shape = jax.ShapeDtypeStruct((1, SQ, D_MODEL), jnp.float32)
    return pl.pallas_call(
        body,
        out_shape=out_shape,
        in_specs=[
            pl.BlockSpec(memory_space=pltpu.VMEM),
            pl.BlockSpec(memory_space=pltpu.VMEM),
            pl.BlockSpec(memory_space=pltpu.MemorySpace.HBM),
            pl.BlockSpec(memory_space=pltpu.MemorySpace.HBM),
            pl.BlockSpec(memory_space=pltpu.VMEM),
        ],
        out_specs=pl.BlockSpec(memory_space=pltpu.VMEM),
        scratch_shapes=[
            pltpu.VMEM((N_DEV, SQ, D_MODEL), jnp.bfloat16),
            pltpu.VMEM((N_DEV, SQ, D_MODEL), jnp.bfloat16),
            pltpu.VMEM((SQ, H * DH), jnp.bfloat16),
            pltpu.VMEM((SQ, H * DH), jnp.bfloat16),
            pltpu.VMEM((SQ, H * DH), jnp.bfloat16),
            pltpu.VMEM((SQ, SKV), jnp.float32),
            pltpu.VMEM((N_DEV - 1, SQ, D_MODEL), jnp.bfloat16),
            pltpu.VMEM((H, SKV, DH), jnp.float32),
            pltpu.VMEM((H, SKV, DH), jnp.float32),
            pltpu.SemaphoreType.DMA((2,)),
            pltpu.SemaphoreType.DMA((2,)),
            pltpu.SemaphoreType.DMA((1,)),
            pltpu.SemaphoreType.DMA((1,)),
            pltpu.SemaphoreType.DMA((N_DEV - 1,)),
            pltpu.SemaphoreType.DMA((N_DEV - 1,)),
            pltpu.SemaphoreType.DMA((H,)),
            pltpu.SemaphoreType.DMA((H,)),
        ],
        compiler_params=pltpu.CompilerParams(
            collective_id=0, vmem_limit_bytes=100 * 1024 * 1024),
    )(xb, wq_b, K_ext, V_ext, wo_b)


# device time: 137741 ns/iter; 2.6960x vs baseline; 1.0073x over previous
import jax
import jax.numpy as jnp
from jax import lax
from jax.experimental import pallas as pl
from jax.experimental.pallas import tpu as pltpu

N_DEV = 4
SQ = 1024
SKV = 1024
H = 8
DH = 128
D_MODEL = 1024
SCALE = 0.08838834764831843
LOG2E = 1.4426950408889634
QT = 128
KT = SKV // 128


def kernel(x, Wq, K_ext, V_ext, Wo):
    i = lax.axis_index("i")

    xb = x.astype(jnp.bfloat16)
    wq_b = (Wq * (SCALE * LOG2E)).astype(jnp.bfloat16)
    wo_b = Wo.astype(jnp.bfloat16)


    def _segments(t):
        if t == 0:
            return [(0, KT)]
        lo = (QT * t - 128) // 128
        hi = min((QT * (t + 1) - 1 + 128) // 128, KT - 1)
        if lo <= 1:
            return [(0, hi + 1)]
        return [(0, 1), (lo, hi + 1)]

    def body(x_ref, wq_ref, k_hbm, v_hbm, wo_ref, out_ref,
             x_full, acc, qall, q0buf, ctx_all, maskadd, rs_recv, kbuf, vbuf,
             ag_ss, ag_rs, al_ss, al_rs, rs_ss, rs_rs, k_sems, v_sems):
        my = lax.axis_index("i")
        left = (my + N_DEV - 1) % N_DEV
        right = (my + 1) % N_DEV

        barrier = pltpu.get_barrier_semaphore()
        pl.semaphore_signal(barrier, inc=1, device_id=(left,),
                            device_id_type=pl.DeviceIdType.MESH)
        pl.semaphore_signal(barrier, inc=1, device_id=(right,),
                            device_id_type=pl.DeviceIdType.MESH)
        pl.semaphore_wait(barrier, 2)

        ag = [
            pltpu.make_async_remote_copy(
                src_ref=x_ref.at[0],
                dst_ref=x_full.at[3],
                send_sem=ag_ss.at[0],
                recv_sem=ag_rs.at[0],
                device_id=(right,),
                device_id_type=pl.DeviceIdType.MESH,
            ),
            pltpu.make_async_remote_copy(
                src_ref=x_full.at[1],
                dst_ref=x_full.at[2],
                send_sem=ag_ss.at[1],
                recv_sem=ag_rs.at[1],
                device_id=(left,),
                device_id_type=pl.DeviceIdType.MESH,
            ),
        ]
        ag_l = pltpu.make_async_remote_copy(
            src_ref=x_ref.at[0],
            dst_ref=x_full.at[1],
            send_sem=al_ss.at[0],
            recv_sem=al_rs.at[0],
            device_id=(left,),
            device_id_type=pl.DeviceIdType.MESH,
        )
        rs_c = pltpu.make_async_remote_copy(
            src_ref=acc.at[3], dst_ref=rs_recv.at[0],
            send_sem=rs_ss.at[0], recv_sem=rs_rs.at[0],
            device_id=(left,), device_id_type=pl.DeviceIdType.MESH,
        )
        rs_b = pltpu.make_async_remote_copy(
            src_ref=acc.at[2], dst_ref=rs_recv.at[1],
            send_sem=rs_ss.at[1], recv_sem=rs_rs.at[1],
            device_id=(right,), device_id_type=pl.DeviceIdType.MESH,
        )
        rs_a = pltpu.make_async_remote_copy(
            src_ref=acc.at[1], dst_ref=rs_recv.at[2],
            send_sem=rs_ss.at[2], recv_sem=rs_rs.at[2],
            device_id=(right,), device_id_type=pl.DeviceIdType.MESH,
        )

        def kv_copies(j):
            b = (my + j) % N_DEV
            out = []
            for h in range(H):
                hh = H * my + h
                out.append(pltpu.make_async_copy(
                    k_hbm.at[b, :, hh, :], kbuf.at[h], k_sems.at[h]))
                out.append(pltpu.make_async_copy(
                    v_hbm.at[b, :, hh, :], vbuf.at[h], v_sems.at[h]))
            return out

        def proj_q(j, qdst):
            x_j = x_ref[0] if j == 0 else x_full[j]
            qdst[:, :] = jnp.dot(x_j, wq_ref[:, :],
                                 preferred_element_type=jnp.float32
                                 ).astype(jnp.bfloat16)

        def attn_out(j, qsrc, next_j=None):
            for c in kv_copies(j):
                c.wait()

            def head_body(h, _):
                off = pl.multiple_of(h * DH, DH)
                q = qsrc[:, pl.ds(off, DH)]
                kh = kbuf[pl.ds(h, 1), :, :].reshape(SKV, DH).astype(
                    jnp.bfloat16)
                vh = vbuf[pl.ds(h, 1), :, :].reshape(SKV, DH).astype(
                    jnp.bfloat16)
                for tq in range(SQ // QT):
                    r0, r1 = tq * QT, (tq + 1) * QT
                    qt = q[r0:r1, :]
                    ctx_t = None
                    den = None
                    for (c0, c1) in _segments(tq):
                        k0, k1 = c0 * 128, c1 * 128
                        s = lax.dot_general(
                            qt, kh[k0:k1, :], (((1,), (1,)), ((), ())),
                            preferred_element_type=jnp.float32)
                        w = jnp.exp2(s + maskadd[r0:r1, k0:k1])
                        dseg = jnp.sum(w, axis=1, keepdims=True)
                        cseg = jnp.dot(w.astype(jnp.bfloat16), vh[k0:k1, :],
                                       preferred_element_type=jnp.float32)
                        ctx_t = cseg if ctx_t is None else ctx_t + cseg
                        den = dseg if den is None else den + dseg
                    ctx_all[r0:r1, pl.ds(off, DH)] = (
                        ctx_t * (1.0 / den)).astype(jnp.bfloat16)
                return 0

            lax.fori_loop(0, H, head_body, 0, unroll=2)
            if next_j is not None:
                for c in kv_copies(next_j):
                    c.start()
            acc[j] = jnp.dot(ctx_all[:, :], wo_ref[:, :],
                             preferred_element_type=jnp.float32
                             ).astype(jnp.bfloat16)

        def rs_add(s):
            tgt = N_DEV - 2 - s
            acc[tgt] = (acc[tgt].astype(jnp.float32)
                        + rs_recv[s].astype(jnp.float32)
                        ).astype(jnp.bfloat16)

        def compute_slot(j, next_j=None):
            proj_q(j, qall)
            attn_out(j, qall, next_j)

        for c in kv_copies(3):
            c.start()
        ag[0].start()
        ag_l.start()
        qi = lax.broadcasted_iota(jnp.int32, (SQ, SKV), 0)
        ki = lax.broadcasted_iota(jnp.int32, (SQ, SKV), 1)
        keep = (jnp.abs(qi - ki) <= 128) | (ki < 32) | (qi < 32)
        maskadd[:, :] = jnp.where(keep, 0.0, -1e9).astype(jnp.float32)

        proj_q(0, q0buf)
        ag_l.wait_recv()
        ag[1].start()
        ag[0].wait_recv()
        compute_slot(3, next_j=2)
        rs_c.start()
        ag[1].wait_recv()
        compute_slot(2, next_j=1)
        rs_b.start()
        compute_slot(1, next_j=0)
        rs_b.wait_recv()
        acc[1] = (acc[1].astype(jnp.float32)
                  + rs_recv[1].astype(jnp.float32)).astype(jnp.bfloat16)
        rs_a.start()
        attn_out(0, q0buf)
        rs_c.wait_recv()
        rs_a.wait_recv()
        out_ref[0] = (acc[0].astype(jnp.float32)
                      + rs_recv[0].astype(jnp.float32)
                      + rs_recv[2].astype(jnp.float32))

        for d in ag:
            d.wait_send()
        ag_l.wait_send()
        for d in (rs_c, rs_b, rs_a):
            d.wait_send()

    out_shape = jax.ShapeDtypeStruct((1, SQ, D_MODEL), jnp.float32)
    return pl.pallas_call(
        body,
        out_shape=out_shape,
        in_specs=[
            pl.BlockSpec(memory_space=pltpu.VMEM),
            pl.BlockSpec(memory_space=pltpu.VMEM),
            pl.BlockSpec(memory_space=pltpu.MemorySpace.HBM),
            pl.BlockSpec(memory_space=pltpu.MemorySpace.HBM),
            pl.BlockSpec(memory_space=pltpu.VMEM),
        ],
        out_specs=pl.BlockSpec(memory_space=pltpu.VMEM),
        scratch_shapes=[
            pltpu.VMEM((N_DEV, SQ, D_MODEL), jnp.bfloat16),
            pltpu.VMEM((N_DEV, SQ, D_MODEL), jnp.bfloat16),
            pltpu.VMEM((SQ, H * DH), jnp.bfloat16),
            pltpu.VMEM((SQ, H * DH), jnp.bfloat16),
            pltpu.VMEM((SQ, H * DH), jnp.bfloat16),
            pltpu.VMEM((SQ, SKV), jnp.float32),
            pltpu.VMEM((N_DEV - 1, SQ, D_MODEL), jnp.bfloat16),
            pltpu.VMEM((H, SKV, DH), jnp.float32),
            pltpu.VMEM((H, SKV, DH), jnp.float32),
            pltpu.SemaphoreType.DMA((2,)),
            pltpu.SemaphoreType.DMA((2,)),
            pltpu.SemaphoreType.DMA((1,)),
            pltpu.SemaphoreType.DMA((1,)),
            pltpu.SemaphoreType.DMA((N_DEV - 1,)),
            pltpu.SemaphoreType.DMA((N_DEV - 1,)),
            pltpu.SemaphoreType.DMA((H,)),
            pltpu.SemaphoreType.DMA((H,)),
        ],
        compiler_params=pltpu.CompilerParams(
            collective_id=0, vmem_limit_bytes=100 * 1024 * 1024),
    )(xb, wq_b, K_ext, V_ext, wo_b)
